# baseline (speedup 1.0000x reference)
import jax
import jax.numpy as jnp
import numpy as np
from jax import lax
from jax.experimental import pallas as pl
from jax.experimental.pallas import tpu as pltpu
from jax.experimental.pallas import tpu_sc as plsc

H = 64
N = 10000
E = 160000
NC = 2
NS = 16
NW = NC * NS
NDEN = 10240
NPADT = NDEN + 16
EPAD = 163840
RPT = NDEN // NW
STRIP = 2048

_BN = float(1.0 / np.sqrt(1.0 + 1e-5))
_SCP = pltpu.CompilerParams(needs_layout_passes=False)


def _lrelu(x, s=0.01):
    return jnp.where(x >= 0, x, s * x)


def _nodes_body(gm_ref, sg_ref, sp_ref, sl_ref, wl_ref, bl_ref, wr_ref,
                br_ref, xl_ref, xr_ref):
    x = gm_ref[...]

    def sub(xin, p_ref):
        w1 = p_ref[0:64, 0:32]
        b1 = p_ref[64:65, 0:32]
        g1 = p_ref[65:66, 0:32]
        be1 = p_ref[66:67, 0:32]
        w2 = p_ref[0:32, 32:96]
        b2 = p_ref[67:68, 32:96]
        g2 = p_ref[68:69, 32:96]
        be2 = p_ref[69:70, 32:96]
        h = jnp.dot(xin, w1, preferred_element_type=jnp.float32) + b1
        h = _lrelu(h * (g1 * _BN) + be1)
        h = jnp.dot(h, w2, preferred_element_type=jnp.float32) + b2
        return _lrelu(h * (g2 * _BN) + be2)

    gm = jnp.concatenate([
        sub(x[:, 0:64], sg_ref),
        sub(x[:, 64:128], sp_ref),
        sub(x[:, 128:192], sl_ref),
    ], axis=1)
    xl_ref[...] = jnp.dot(gm, wl_ref[...],
                          preferred_element_type=jnp.float32) + bl_ref[...]
    xr_ref[...] = jnp.dot(gm, wr_ref[...],
                          preferred_element_type=jnp.float32) + br_ref[...]


def _pack_sub(p):
    buf = jnp.zeros((70, 96), jnp.float32)
    w1 = p["W1"]
    buf = buf.at[0:64, 0:32].set(w1)
    buf = buf.at[64, 0:32].set(p["b1"])
    buf = buf.at[65, 0:32].set(p["g1"])
    buf = buf.at[66, 0:32].set(p["be1"])
    buf = buf.at[0:32, 32:96].set(p["W2"])
    buf = buf.at[67, 32:96].set(p["b2"])
    buf = buf.at[68, 32:96].set(p["g2"])
    buf = buf.at[69, 32:96].set(p["be2"])
    return buf


def _sc_logits_body(xl_hbm, xr_hbm, src_hbm, dst_hbm, att_hbm, zden_hbm,
                    p4_hbm, den_hbm,
                    att_v, slo, dlo, xs, xrow, logit_b, pb0, pb1, pb2, pb3,
                    dtab, sem):
    pbufs = (pb0, pb1, pb2, pb3)
    c = lax.axis_index("c")
    s = lax.axis_index("s")
    wid = c * NS + s
    epw = EPAD // NW
    nchunks = epw // 16

    pltpu.sync_copy(att_hbm, att_v)
    pltpu.sync_copy(zden_hbm, dtab)
    iota = lax.iota(jnp.int32, 16)
    lane0 = (iota == 0).astype(jnp.float32)
    last = iota == 15

    def chunk(i, carry):
        base = wid * epw + i * 16
        pltpu.sync_copy(src_hbm.at[pl.ds(base, 16)], slo)
        pltpu.sync_copy(dst_hbm.at[pl.ds(base, 16)], dlo)
        cp1 = pltpu.async_copy(xl_hbm.at[slo], xs, sem)
        cp2 = pltpu.async_copy(xr_hbm.at[dlo], xrow, sem)
        cp1.wait()
        cp2.wait()
        for e in range(16):
            for h in range(4):
                acc = jnp.zeros((16,), jnp.float32)
                for k in range(4):
                    col = h * 64 + k * 16
                    sv = xs[e, pl.ds(col, 16)] + xrow[e, pl.ds(col, 16)]
                    lr = jnp.where(sv >= 0, sv, 0.2 * sv)
                    acc = acc + lr * att_v[pl.ds(col, 16)]
                cum = jnp.cumsum(acc)
                plsc.store_scatter(
                    logit_b,
                    [jnp.full((16,), h, jnp.int32),
                     jnp.full((16,), e, jnp.int32)],
                    cum, mask=last)
        dv = dlo[...]
        pvs = []
        for h in range(4):
            pv = jnp.exp(logit_b[h, :])
            pbufs[h][...] = pv
            pvs.append(pv)
            pltpu.sync_copy(pbufs[h], p4_hbm.at[pl.ds(h * EPAD + base, 16)])
        for e in range(16):
            d = dv[e]
            for h in range(4):
                off = h * NDEN + d
                cur = dtab[pl.ds(off, 16)]
                dtab[pl.ds(off, 16)] = cur + pvs[h][e] * lane0
        return carry

    lax.fori_loop(0, nchunks, chunk, None)
    pltpu.sync_copy(dtab, den_hbm.at[wid, 0])


def _denom_body(den_ref, out_ref):
    acc = den_ref[0]
    for i in range(1, NW):
        acc = acc + den_ref[i]
    inv = 1.0 / (2.0 * acc + 1e-16)
    out_ref[...] = jnp.concatenate([inv, inv], axis=0)


def _sc_alpha_body(p4_hbm, dinv_hbm, dst_hbm, attn_hbm,
                   den0, den1, den2, den3, dst_b, pv_b, ab, sem):
    dens = (den0, den1, den2, den3)
    c = lax.axis_index("c")
    s = lax.axis_index("s")
    wid = c * NS + s
    epw = EPAD // NW
    nchunks = epw // 16

    for h in range(4):
        pltpu.sync_copy(dinv_hbm.at[pl.ds(h * NDEN, NDEN)], dens[h])

    def chunk(i, carry):
        base = wid * epw + i * 16
        pltpu.sync_copy(dst_hbm.at[pl.ds(base, 16)], dst_b)
        dv = dst_b[...]
        for h in range(4):
            pltpu.sync_copy(p4_hbm.at[pl.ds(h * EPAD + base, 16)], pv_b)
            dinv = plsc.load_gather(dens[h], [dv])
            ab[...] = pv_b[...] * dinv
            pltpu.sync_copy(ab, attn_hbm.at[pl.ds(h * EPAD + base, 16)])
        return carry

    lax.fori_loop(0, nchunks, chunk, None)


def _sc_aggregate_body(xl_hbm, al_hbm, src_hbm, dst_hbm, zacc_hbm,
                       out_hbm,
                       dstrip, sstrip, a0s, a1s, a2s, a3s,
                       ls, ld, l0, l1, l2, l3, idx_b, rows, acc, sem):
    c = lax.axis_index("c")
    s = lax.axis_index("s")
    wid = c * NS + s
    lo = wid * RPT
    hi = lo + RPT
    nstrips = EPAD // STRIP

    pltpu.sync_copy(zacc_hbm, acc)

    def strip_fn(t, carry):
        sbase = t * STRIP
        pltpu.sync_copy(dst_hbm.at[pl.ds(sbase, STRIP)], dstrip)
        pltpu.sync_copy(src_hbm.at[pl.ds(sbase, STRIP)], sstrip)
        pltpu.sync_copy(al_hbm.at[pl.ds(0 * EPAD + sbase, STRIP)], a0s)
        pltpu.sync_copy(al_hbm.at[pl.ds(1 * EPAD + sbase, STRIP)], a1s)
        pltpu.sync_copy(al_hbm.at[pl.ds(2 * EPAD + sbase, STRIP)], a2s)
        pltpu.sync_copy(al_hbm.at[pl.ds(3 * EPAD + sbase, STRIP)], a3s)

        def grp(j, cnt):
            dv = dstrip[pl.ds(j * 16, 16)]
            m = jnp.logical_and(dv >= lo, dv < hi)
            npc = plsc.all_reduce_population_count(m)[0]

            @pl.when(npc > 0)
            def _():
                plsc.store_compressed(ld.at[pl.ds(cnt, 16)], dv, mask=m)
                plsc.store_compressed(ls.at[pl.ds(cnt, 16)],
                                      sstrip[pl.ds(j * 16, 16)], mask=m)
                plsc.store_compressed(l0.at[pl.ds(cnt, 16)],
                                      a0s[pl.ds(j * 16, 16)], mask=m)
                plsc.store_compressed(l1.at[pl.ds(cnt, 16)],
                                      a1s[pl.ds(j * 16, 16)], mask=m)
                plsc.store_compressed(l2.at[pl.ds(cnt, 16)],
                                      a2s[pl.ds(j * 16, 16)], mask=m)
                plsc.store_compressed(l3.at[pl.ds(cnt, 16)],
                                      a3s[pl.ds(j * 16, 16)], mask=m)

            return cnt + npc

        cnt = lax.fori_loop(0, STRIP // 16, grp, jnp.int32(0))
        ld[pl.ds(cnt, 16)] = jnp.full((16,), lo, jnp.int32)
        ls[pl.ds(cnt, 16)] = jnp.zeros((16,), jnp.int32)
        zf = jnp.zeros((16,), jnp.float32)
        l0[pl.ds(cnt, 16)] = zf
        l1[pl.ds(cnt, 16)] = zf
        l2[pl.ds(cnt, 16)] = zf
        l3[pl.ds(cnt, 16)] = zf

        def agg(k, carry2):
            kb = k * 16
            idx_b[...] = ls[pl.ds(kb, 16)]
            pltpu.async_copy(xl_hbm.at[idx_b], rows, sem).wait()
            dm = ld[pl.ds(kb, 16)] - lo
            a0 = l0[pl.ds(kb, 16)]
            a1 = l1[pl.ds(kb, 16)]
            a2 = l2[pl.ds(kb, 16)]
            a3 = l3[pl.ds(kb, 16)]
            for e in range(16):
                d = dm[e]
                rbase = d * 256
                ws = (a0[e], a1[e], a2[e], a3[e])
                for k16 in range(16):
                    off = rbase + k16 * 16
                    cur = acc[pl.ds(off, 16)]
                    acc[pl.ds(off, 16)] = (
                        cur + rows[e, pl.ds(k16 * 16, 16)] * ws[k16 // 4])
            return carry2

        lax.fori_loop(0, (cnt + 15) // 16, agg, None)
        return carry

    lax.fori_loop(0, nstrips, strip_fn, None)
    pltpu.sync_copy(acc, out_hbm.at[wid, 0])


def _nodepost_body(acc_ref, bias_ref, pw_ref, pb_ref, pg_ref, pbe_ref,
                   rw1_ref, qn_ref):
    h1 = 2.0 * acc_ref[...] + bias_ref[...]
    p = jnp.dot(h1, pw_ref[...], preferred_element_type=jnp.float32) + pb_ref[...]
    p = _lrelu(p * (pg_ref[...] * _BN) + pbe_ref[...])
    qn_ref[...] = jnp.dot(p, rw1_ref[...], preferred_element_type=jnp.float32)


def _sc_gather_body(qn_hbm, src_hbm, dst_hbm, g_hbm,
                    src_b, dst_b, rs, rd, gsum, sem):
    c = lax.axis_index("c")
    s = lax.axis_index("s")
    wid = c * NS + s
    epw = EPAD // NW
    nchunks = epw // 32

    def chunk(i, carry):
        base = wid * epw + i * 32
        pltpu.sync_copy(src_hbm.at[pl.ds(base, 32)], src_b)
        pltpu.sync_copy(dst_hbm.at[pl.ds(base, 32)], dst_b)
        cp1 = pltpu.async_copy(qn_hbm.at[src_b], rs, sem)
        cp2 = pltpu.async_copy(qn_hbm.at[dst_b], rd, sem)
        cp1.wait()
        cp2.wait()
        for e in range(32):
            for k in range(8):
                gsum[e, pl.ds(k * 16, 16)] = (rs[e, pl.ds(k * 16, 16)]
                                              + rd[e, pl.ds(k * 16, 16)])
        pltpu.sync_copy(gsum, g_hbm.at[pl.ds(base, 32)])
        return carry

    lax.fori_loop(0, nchunks, chunk, None)


def _edge_body(g_ref, x_ref, rb1_ref, rg1_ref, rbe1_ref, rw2_ref, rb2_ref,
               rg2_ref, rbe2_ref, cp_ref, cl_ref, mth_ref, fw1_ref, fb1_ref,
               fg1_ref, fbe1_ref, fw2_ref, fb2_ref, fg2_ref, fbe2_ref,
               fw3_ref, fb3_ref, o_ref):
    g = g_ref[...]
    h = _lrelu((g + rb1_ref[...]) * (rg1_ref[...] * _BN) + rbe1_ref[...])
    h = jnp.dot(h, rw2_ref[...], preferred_element_type=jnp.float32) + rb2_ref[...]
    preds = _lrelu(h * (rg2_ref[...] * _BN) + rbe2_ref[...])

    x = x_ref[...]

    def co(xcol, p_ref):
        w1 = p_ref[0:1, 0:32]
        b1 = p_ref[1:2, 0:32]
        g1 = p_ref[2:3, 0:32]
        be1 = p_ref[3:4, 0:32]
        w2 = p_ref[4:36, 32:96]
        b2 = p_ref[1:2, 32:96]
        g2 = p_ref[2:3, 32:96]
        be2 = p_ref[3:4, 32:96]
        hh = xcol * w1 + b1
        hh = _lrelu(hh * (g1 * _BN) + be1)
        hh = jnp.dot(hh, w2, preferred_element_type=jnp.float32) + b2
        return _lrelu(hh * (g2 * _BN) + be2)

    ip = co(x[:, 0:1], cp_ref)
    il = co(x[:, 1:2], cl_ref)

    mw1 = mth_ref[0:14, 0:32]
    mb1 = mth_ref[14:15, 0:32]
    mg1 = mth_ref[15:16, 0:32]
    mbe1 = mth_ref[16:17, 0:32]
    mw2 = mth_ref[0:32, 32:96]
    mb2 = mth_ref[17:18, 32:96]
    mg2 = mth_ref[18:19, 32:96]
    mbe2 = mth_ref[19:20, 32:96]
    hm = jnp.dot(x[:, 2:16], mw1, preferred_element_type=jnp.float32) + mb1
    hm = _lrelu(hm * (mg1 * _BN) + mbe1)
    hm = jnp.dot(hm, mw2, preferred_element_type=jnp.float32) + mb2
    im = _lrelu(hm * (mg2 * _BN) + mbe2)

    fw1 = fw1_ref[...]
    y = (jnp.dot(preds, fw1[0:64], preferred_element_type=jnp.float32)
         + jnp.dot(ip, fw1[64:128], preferred_element_type=jnp.float32)
         + jnp.dot(il, fw1[128:192], preferred_element_type=jnp.float32)
         + jnp.dot(im, fw1[192:256], preferred_element_type=jnp.float32)
         + fb1_ref[...])
    h = _lrelu(y * (fg1_ref[...] * _BN) + fbe1_ref[...])
    h = jnp.dot(h, fw2_ref[...], preferred_element_type=jnp.float32) + fb2_ref[...]
    h = _lrelu(h * (fg2_ref[...] * _BN) + fbe2_ref[...])
    sg = jax.nn.sigmoid((h * fw3_ref[...]).sum(-1, keepdims=True) + fb3_ref[0, 0])
    o_ref[...] = jnp.broadcast_to(sg, (sg.shape[0], 8))


def _pack_co(p):
    buf = jnp.zeros((36, 96), jnp.float32)
    buf = buf.at[0, 0:32].set(p["W1"][0])
    buf = buf.at[1, 0:32].set(p["b1"])
    buf = buf.at[2, 0:32].set(p["g1"])
    buf = buf.at[3, 0:32].set(p["be1"])
    buf = buf.at[4:36, 32:96].set(p["W2"])
    buf = buf.at[1, 32:96].set(p["b2"])
    buf = buf.at[2, 32:96].set(p["g2"])
    buf = buf.at[3, 32:96].set(p["be2"])
    return buf


def _pack_meth(p):
    buf = jnp.zeros((32, 96), jnp.float32)
    buf = buf.at[0:14, 0:32].set(p["W1"])
    buf = buf.at[14, 0:32].set(p["b1"])
    buf = buf.at[15, 0:32].set(p["g1"])
    buf = buf.at[16, 0:32].set(p["be1"])
    buf = buf.at[0:32, 32:96].set(p["W2"])
    buf = buf.at[17, 32:96].set(p["b2"])
    buf = buf.at[18, 32:96].set(p["g2"])
    buf = buf.at[19, 32:96].set(p["be2"])
    return buf


def kernel(interaction, edge_index, graph_matrix, params):
    pg = params["gat"]
    full = lambda i: (0, 0)

    xlf, xrf = pl.pallas_call(
        _nodes_body,
        grid=(5,),
        in_specs=[
            pl.BlockSpec((2000, 192), lambda i: (i, 0)),
            pl.BlockSpec((70, 96), full),
            pl.BlockSpec((70, 96), full),
            pl.BlockSpec((70, 96), full),
            pl.BlockSpec((192, 256), full),
            pl.BlockSpec((1, 256), full),
            pl.BlockSpec((192, 256), full),
            pl.BlockSpec((1, 256), full),
        ],
        out_specs=[
            pl.BlockSpec((2000, 256), lambda i: (i, 0)),
            pl.BlockSpec((2000, 256), lambda i: (i, 0)),
        ],
        out_shape=[
            jax.ShapeDtypeStruct((N, 256), jnp.float32),
            jax.ShapeDtypeStruct((N, 256), jnp.float32),
        ],
    )(graph_matrix, _pack_sub(params["sub_g"]), _pack_sub(params["sub_p"]),
      _pack_sub(params["sub_l"]), pg["Wl"], pg["bl"].reshape(1, 256),
      pg["Wr"], pg["br"].reshape(1, 256))

    src = edge_index[:, 0]
    dst = edge_index[:, 1]
    srcp = jnp.concatenate([src, jnp.zeros((EPAD - E,), jnp.int32)])
    dstp = jnp.concatenate([dst, jnp.full((EPAD - E,), N, jnp.int32)])

    mesh = plsc.VectorSubcoreMesh(core_axis_name="c", subcore_axis_name="s",
                                  num_cores=NC, num_subcores=NS)

    p4, den32 = pl.kernel(
        _sc_logits_body,
        out_type=(
            jax.ShapeDtypeStruct((4 * EPAD,), jnp.float32),
            jax.ShapeDtypeStruct((NW, 1, 4 * NDEN), jnp.float32),
        ),
        mesh=mesh,
        compiler_params=_SCP,
        scratch_types=[
            pltpu.VMEM((256,), jnp.float32),
            pltpu.VMEM((16,), jnp.int32),
            pltpu.VMEM((16,), jnp.int32),
            pltpu.VMEM((16, 256), jnp.float32),
            pltpu.VMEM((16, 256), jnp.float32),
            pltpu.VMEM((4, 16), jnp.float32),
            pltpu.VMEM((16,), jnp.float32),
            pltpu.VMEM((16,), jnp.float32),
            pltpu.VMEM((16,), jnp.float32),
            pltpu.VMEM((16,), jnp.float32),
            pltpu.VMEM((4 * NDEN,), jnp.float32),
            pltpu.SemaphoreType.DMA,
        ],
    )(xlf, xrf, srcp, dstp, pg["att"].reshape(256),
      jnp.zeros((4 * NDEN,), jnp.float32))

    dinv = pl.pallas_call(
        _denom_body,
        grid=(1,),
        in_specs=[pl.BlockSpec((NW, 4, NDEN), lambda i: (0, 0, 0))],
        out_specs=pl.BlockSpec((8, NDEN), lambda i: (0, 0)),
        out_shape=jax.ShapeDtypeStruct((8, NDEN), jnp.float32),
    )(den32.reshape(NW, 4, NDEN))

    attn4 = pl.kernel(
        _sc_alpha_body,
        out_type=jax.ShapeDtypeStruct((4 * EPAD,), jnp.float32),
        mesh=mesh,
        compiler_params=_SCP,
        scratch_types=[
            pltpu.VMEM((NDEN,), jnp.float32),
            pltpu.VMEM((NDEN,), jnp.float32),
            pltpu.VMEM((NDEN,), jnp.float32),
            pltpu.VMEM((NDEN,), jnp.float32),
            pltpu.VMEM((16,), jnp.int32),
            pltpu.VMEM((16,), jnp.float32),
            pltpu.VMEM((16,), jnp.float32),
            pltpu.SemaphoreType.DMA,
        ],
    )(p4, dinv.reshape(8 * NDEN), dstp)

    acc32 = pl.kernel(
        _sc_aggregate_body,
        out_type=jax.ShapeDtypeStruct((NW, 1, RPT * 256), jnp.float32),
        mesh=mesh,
        compiler_params=_SCP,
        scratch_types=[
            pltpu.VMEM((STRIP,), jnp.int32),
            pltpu.VMEM((STRIP,), jnp.int32),
            pltpu.VMEM((STRIP,), jnp.float32),
            pltpu.VMEM((STRIP,), jnp.float32),
            pltpu.VMEM((STRIP,), jnp.float32),
            pltpu.VMEM((STRIP,), jnp.float32),
            pltpu.VMEM((STRIP + 16,), jnp.int32),
            pltpu.VMEM((STRIP + 16,), jnp.int32),
            pltpu.VMEM((STRIP + 16,), jnp.float32),
            pltpu.VMEM((STRIP + 16,), jnp.float32),
            pltpu.VMEM((STRIP + 16,), jnp.float32),
            pltpu.VMEM((STRIP + 16,), jnp.float32),
            pltpu.VMEM((16,), jnp.int32),
            pltpu.VMEM((16, 256), jnp.float32),
            pltpu.VMEM((RPT * 256,), jnp.float32),
            pltpu.SemaphoreType.DMA,
        ],
    )(xlf, attn4, srcp, dstp, jnp.zeros((RPT * 256,), jnp.float32))

    qn = pl.pallas_call(
        _nodepost_body,
        grid=(2,),
        in_specs=[
            pl.BlockSpec((NDEN // 2, 256), lambda i: (i, 0)),
            pl.BlockSpec((1, 256), full),
            pl.BlockSpec((256, 128), full),
            pl.BlockSpec((1, 128), full),
            pl.BlockSpec((1, 128), full),
            pl.BlockSpec((1, 128), full),
            pl.BlockSpec((128, 128), full),
        ],
        out_specs=pl.BlockSpec((NDEN // 2, 128), lambda i: (i, 0)),
        out_shape=jax.ShapeDtypeStruct((NDEN, 128), jnp.float32),
    )(acc32.reshape(NDEN, 256), pg["bias"].reshape(1, 256),
      params["phi1"]["W"], params["phi1"]["b"].reshape(1, 128),
      params["phi1"]["g"].reshape(1, 128),
      params["phi1"]["be"].reshape(1, 128), params["rho1"]["W1"])

    g_edges = pl.kernel(
        _sc_gather_body,
        out_type=jax.ShapeDtypeStruct((EPAD, 128), jnp.float32),
        mesh=mesh,
        compiler_params=_SCP,
        scratch_types=[
            pltpu.VMEM((32,), jnp.int32),
            pltpu.VMEM((32,), jnp.int32),
            pltpu.VMEM((32, 128), jnp.float32),
            pltpu.VMEM((32, 128), jnp.float32),
            pltpu.VMEM((32, 128), jnp.float32),
            pltpu.SemaphoreType.DMA,
        ],
    )(qn, srcp, dstp)

    pr = params["rho1"]
    pf = params["fc"]
    preds8 = pl.pallas_call(
        _edge_body,
        grid=(125,),
        in_specs=[
            pl.BlockSpec((1280, 128), lambda i: (i, 0)),
            pl.BlockSpec((1280, 16), lambda i: (i, 0)),
            pl.BlockSpec((1, 128), full),
            pl.BlockSpec((1, 128), full),
            pl.BlockSpec((1, 128), full),
            pl.BlockSpec((128, 64), full),
            pl.BlockSpec((1, 64), full),
            pl.BlockSpec((1, 64), full),
            pl.BlockSpec((1, 64), full),
            pl.BlockSpec((36, 96), full),
            pl.BlockSpec((36, 96), full),
            pl.BlockSpec((32, 96), full),
            pl.BlockSpec((256, 64), full),
            pl.BlockSpec((1, 64), full),
            pl.BlockSpec((1, 64), full),
            pl.BlockSpec((1, 64), full),
            pl.BlockSpec((64, 32), full),
            pl.BlockSpec((1, 32), full),
            pl.BlockSpec((1, 32), full),
            pl.BlockSpec((1, 32), full),
            pl.BlockSpec((1, 32), full),
            pl.BlockSpec((1, 1), full),
        ],
        out_specs=pl.BlockSpec((1280, 8), lambda i: (i, 0)),
        out_shape=jax.ShapeDtypeStruct((E, 8), jnp.float32),
    )(g_edges[:E], interaction,
      pr["b1"].reshape(1, 128), pr["g1"].reshape(1, 128),
      pr["be1"].reshape(1, 128), pr["W2"], pr["b2"].reshape(1, 64),
      pr["g2"].reshape(1, 64), pr["be2"].reshape(1, 64),
      _pack_co(params["co_p"]), _pack_co(params["co_loc"]),
      _pack_meth(params["meth"]),
      pf["W1"], pf["b1"].reshape(1, 64), pf["g1"].reshape(1, 64),
      pf["be1"].reshape(1, 64), pf["W2"], pf["b2"].reshape(1, 32),
      pf["g2"].reshape(1, 32), pf["be2"].reshape(1, 32),
      pf["W3"].reshape(1, 32), pf["b3"].reshape(1, 1))

    alpha = attn4.reshape(4, EPAD)[:, :E].T
    attn = jnp.concatenate([alpha, alpha], axis=0)
    return preds8[:, 0], attn


# speedup vs baseline: 1.0499x; 1.0499x over previous
import jax
import jax.numpy as jnp
import numpy as np
from jax import lax
from jax.experimental import pallas as pl
from jax.experimental.pallas import tpu as pltpu
from jax.experimental.pallas import tpu_sc as plsc

H = 64
N = 10000
E = 160000
NC = 2
NS = 16
NW = NC * NS
NDEN = 10240
NPADT = NDEN + 16
EPAD = 163840
RPT = NDEN // NW
STRIP = 2048

_BN = float(1.0 / np.sqrt(1.0 + 1e-5))
_SCP = pltpu.CompilerParams(needs_layout_passes=False)


def _lrelu(x, s=0.01):
    return jnp.where(x >= 0, x, s * x)


def _nodes_body(gm_ref, sg_ref, sp_ref, sl_ref, wl_ref, bl_ref, wr_ref,
                br_ref, xl_ref, xr_ref):
    x = gm_ref[...]

    def sub(xin, p_ref):
        w1 = p_ref[0:64, 0:32]
        b1 = p_ref[64:65, 0:32]
        g1 = p_ref[65:66, 0:32]
        be1 = p_ref[66:67, 0:32]
        w2 = p_ref[0:32, 32:96]
        b2 = p_ref[67:68, 32:96]
        g2 = p_ref[68:69, 32:96]
        be2 = p_ref[69:70, 32:96]
        h = jnp.dot(xin, w1, preferred_element_type=jnp.float32) + b1
        h = _lrelu(h * (g1 * _BN) + be1)
        h = jnp.dot(h, w2, preferred_element_type=jnp.float32) + b2
        return _lrelu(h * (g2 * _BN) + be2)

    gm = jnp.concatenate([
        sub(x[:, 0:64], sg_ref),
        sub(x[:, 64:128], sp_ref),
        sub(x[:, 128:192], sl_ref),
    ], axis=1)
    xl_ref[...] = jnp.dot(gm, wl_ref[...],
                          preferred_element_type=jnp.float32) + bl_ref[...]
    xr_ref[...] = jnp.dot(gm, wr_ref[...],
                          preferred_element_type=jnp.float32) + br_ref[...]


def _pack_sub(p):
    buf = jnp.zeros((70, 96), jnp.float32)
    w1 = p["W1"]
    buf = buf.at[0:64, 0:32].set(w1)
    buf = buf.at[64, 0:32].set(p["b1"])
    buf = buf.at[65, 0:32].set(p["g1"])
    buf = buf.at[66, 0:32].set(p["be1"])
    buf = buf.at[0:32, 32:96].set(p["W2"])
    buf = buf.at[67, 32:96].set(p["b2"])
    buf = buf.at[68, 32:96].set(p["g2"])
    buf = buf.at[69, 32:96].set(p["be2"])
    return buf


def _sc_logits_body(xl_hbm, xr_hbm, src_hbm, dst_hbm, att_hbm, zden_hbm,
                    p4_hbm, den_hbm,
                    att_v, slo, dlo, xs, xrow, logit_b, pb0, pb1, pb2, pb3,
                    dtab, sem):
    pbufs = (pb0, pb1, pb2, pb3)
    c = lax.axis_index("c")
    s = lax.axis_index("s")
    wid = c * NS + s
    epw = EPAD // NW
    nchunks = epw // 16

    pltpu.sync_copy(att_hbm, att_v)
    pltpu.sync_copy(zden_hbm, dtab)
    iota = lax.iota(jnp.int32, 16)
    lane0 = (iota == 0).astype(jnp.float32)
    last = iota == 15

    def chunk(i, carry):
        base = wid * epw + i * 16
        pltpu.sync_copy(src_hbm.at[pl.ds(base, 16)], slo)
        pltpu.sync_copy(dst_hbm.at[pl.ds(base, 16)], dlo)
        cp1 = pltpu.async_copy(xl_hbm.at[slo], xs, sem)
        cp2 = pltpu.async_copy(xr_hbm.at[dlo], xrow, sem)
        cp1.wait()
        cp2.wait()
        for e in range(16):
            for h in range(4):
                acc = jnp.zeros((16,), jnp.float32)
                for k in range(4):
                    col = h * 64 + k * 16
                    sv = xs[e, pl.ds(col, 16)] + xrow[e, pl.ds(col, 16)]
                    lr = jnp.where(sv >= 0, sv, 0.2 * sv)
                    acc = acc + lr * att_v[pl.ds(col, 16)]
                cum = jnp.cumsum(acc)
                plsc.store_scatter(
                    logit_b,
                    [jnp.full((16,), h, jnp.int32),
                     jnp.full((16,), e, jnp.int32)],
                    cum, mask=last)
        dv = dlo[...]
        pvs = []
        for h in range(4):
            pv = jnp.exp(logit_b[h, :])
            pbufs[h][...] = pv
            pvs.append(pv)
            pltpu.sync_copy(pbufs[h], p4_hbm.at[pl.ds(h * EPAD + base, 16)])
        for e in range(16):
            d = dv[e]
            for h in range(4):
                off = h * NDEN + d
                cur = dtab[pl.ds(off, 16)]
                dtab[pl.ds(off, 16)] = cur + pvs[h][e] * lane0
        return carry

    lax.fori_loop(0, nchunks, chunk, None)
    pltpu.sync_copy(dtab, den_hbm.at[wid, 0])


def _denom_body(den_ref, out_ref):
    acc = den_ref[0]
    for i in range(1, NW):
        acc = acc + den_ref[i]
    inv = 1.0 / (2.0 * acc + 1e-16)
    out_ref[...] = jnp.concatenate([inv, inv], axis=0)


def _sc_alpha_body(p4_hbm, dinv_hbm, dst_hbm, attn_hbm,
                   den0, den1, den2, den3, dst_b, pv_b, ab, sem):
    dens = (den0, den1, den2, den3)
    c = lax.axis_index("c")
    s = lax.axis_index("s")
    wid = c * NS + s
    epw = EPAD // NW
    nchunks = epw // 16

    for h in range(4):
        pltpu.sync_copy(dinv_hbm.at[pl.ds(h * NDEN, NDEN)], dens[h])

    def chunk(i, carry):
        base = wid * epw + i * 16
        pltpu.sync_copy(dst_hbm.at[pl.ds(base, 16)], dst_b)
        dv = dst_b[...]
        for h in range(4):
            pltpu.sync_copy(p4_hbm.at[pl.ds(h * EPAD + base, 16)], pv_b)
            dinv = plsc.load_gather(dens[h], [dv])
            ab[...] = pv_b[...] * dinv
            pltpu.sync_copy(ab, attn_hbm.at[pl.ds(h * EPAD + base, 16)])
        return carry

    lax.fori_loop(0, nchunks, chunk, None)


def _sc_aggregate_body(xl_hbm, al_hbm, src_hbm, dst_hbm, zacc_hbm,
                       out_hbm,
                       dstrip, sstrip,
                       ls, ld, lidx, idx_b, ib0, ib1, ib2, ib3,
                       av0, av1, av2, av3, rows, acc, sem):
    c = lax.axis_index("c")
    s = lax.axis_index("s")
    wid = c * NS + s
    lo = wid * RPT
    hi = lo + RPT
    nstrips = EPAD // STRIP
    iota = lax.iota(jnp.int32, 16)

    pltpu.sync_copy(zacc_hbm, acc)

    def strip_fn(t, carry):
        sbase = t * STRIP
        pltpu.sync_copy(dst_hbm.at[pl.ds(sbase, STRIP)], dstrip)
        pltpu.sync_copy(src_hbm.at[pl.ds(sbase, STRIP)], sstrip)

        def grp(j, cnt):
            dv = dstrip[pl.ds(j * 16, 16)]
            m = jnp.logical_and(dv >= lo, dv < hi)
            npc = plsc.all_reduce_population_count(m)[0]

            @pl.when(npc > 0)
            def _():
                plsc.store_compressed(ld.at[pl.ds(cnt, 16)], dv, mask=m)
                plsc.store_compressed(ls.at[pl.ds(cnt, 16)],
                                      sstrip[pl.ds(j * 16, 16)], mask=m)
                plsc.store_compressed(lidx.at[pl.ds(cnt, 16)],
                                      sbase + j * 16 + iota, mask=m)

            return cnt + npc

        cnt = lax.fori_loop(0, STRIP // 16, grp, jnp.int32(0))
        ld[pl.ds(cnt, 16)] = jnp.full((16,), lo, jnp.int32)
        ls[pl.ds(cnt, 16)] = jnp.zeros((16,), jnp.int32)
        lidx[pl.ds(cnt, 16)] = jnp.zeros((16,), jnp.int32)

        def agg(k, carry2):
            kb = k * 16
            idx_b[...] = ls[pl.ds(kb, 16)]
            ev = lidx[pl.ds(kb, 16)]
            ib0[...] = ev
            ib1[...] = ev + EPAD
            ib2[...] = ev + 2 * EPAD
            ib3[...] = ev + 3 * EPAD
            cpr = pltpu.async_copy(xl_hbm.at[idx_b], rows, sem)
            c0 = pltpu.async_copy(al_hbm.at[ib0], av0, sem)
            c1 = pltpu.async_copy(al_hbm.at[ib1], av1, sem)
            c2 = pltpu.async_copy(al_hbm.at[ib2], av2, sem)
            c3 = pltpu.async_copy(al_hbm.at[ib3], av3, sem)
            cpr.wait()
            c0.wait()
            c1.wait()
            c2.wait()
            c3.wait()
            mk = ((kb + iota) < cnt).astype(jnp.float32)
            dm = ld[pl.ds(kb, 16)] - lo
            a0 = av0[...] * mk
            a1 = av1[...] * mk
            a2 = av2[...] * mk
            a3 = av3[...] * mk
            for e in range(16):
                d = dm[e]
                rbase = d * 256
                ws = (a0[e], a1[e], a2[e], a3[e])
                for k16 in range(16):
                    off = rbase + k16 * 16
                    cur = acc[pl.ds(off, 16)]
                    acc[pl.ds(off, 16)] = (
                        cur + rows[e, pl.ds(k16 * 16, 16)] * ws[k16 // 4])
            return carry2

        lax.fori_loop(0, (cnt + 15) // 16, agg, None)
        return carry

    lax.fori_loop(0, nstrips, strip_fn, None)
    pltpu.sync_copy(acc, out_hbm.at[wid, 0])


def _nodepost_body(acc_ref, bias_ref, pw_ref, pb_ref, pg_ref, pbe_ref,
                   rw1_ref, qn_ref):
    h1 = 2.0 * acc_ref[...] + bias_ref[...]
    p = jnp.dot(h1, pw_ref[...], preferred_element_type=jnp.float32) + pb_ref[...]
    p = _lrelu(p * (pg_ref[...] * _BN) + pbe_ref[...])
    qn_ref[...] = jnp.dot(p, rw1_ref[...], preferred_element_type=jnp.float32)


def _sc_gather_body(qn_hbm, src_hbm, dst_hbm, g_hbm,
                    src_b, dst_b, rs, rd, gsum, sem):
    c = lax.axis_index("c")
    s = lax.axis_index("s")
    wid = c * NS + s
    epw = EPAD // NW
    nchunks = epw // 32

    def chunk(i, carry):
        base = wid * epw + i * 32
        pltpu.sync_copy(src_hbm.at[pl.ds(base, 32)], src_b)
        pltpu.sync_copy(dst_hbm.at[pl.ds(base, 32)], dst_b)
        cp1 = pltpu.async_copy(qn_hbm.at[src_b], rs, sem)
        cp2 = pltpu.async_copy(qn_hbm.at[dst_b], rd, sem)
        cp1.wait()
        cp2.wait()
        for e in range(32):
            for k in range(8):
                gsum[e, pl.ds(k * 16, 16)] = (rs[e, pl.ds(k * 16, 16)]
                                              + rd[e, pl.ds(k * 16, 16)])
        pltpu.sync_copy(gsum, g_hbm.at[pl.ds(base, 32)])
        return carry

    lax.fori_loop(0, nchunks, chunk, None)


def _edge_body(g_ref, x_ref, rb1_ref, rg1_ref, rbe1_ref, rw2_ref, rb2_ref,
               rg2_ref, rbe2_ref, cp_ref, cl_ref, mth_ref, fw1_ref, fb1_ref,
               fg1_ref, fbe1_ref, fw2_ref, fb2_ref, fg2_ref, fbe2_ref,
               fw3_ref, fb3_ref, o_ref):
    g = g_ref[...]
    h = _lrelu((g + rb1_ref[...]) * (rg1_ref[...] * _BN) + rbe1_ref[...])
    h = jnp.dot(h, rw2_ref[...], preferred_element_type=jnp.float32) + rb2_ref[...]
    preds = _lrelu(h * (rg2_ref[...] * _BN) + rbe2_ref[...])

    x = x_ref[...]

    def co(xcol, p_ref):
        w1 = p_ref[0:1, 0:32]
        b1 = p_ref[1:2, 0:32]
        g1 = p_ref[2:3, 0:32]
        be1 = p_ref[3:4, 0:32]
        w2 = p_ref[4:36, 32:96]
        b2 = p_ref[1:2, 32:96]
        g2 = p_ref[2:3, 32:96]
        be2 = p_ref[3:4, 32:96]
        hh = xcol * w1 + b1
        hh = _lrelu(hh * (g1 * _BN) + be1)
        hh = jnp.dot(hh, w2, preferred_element_type=jnp.float32) + b2
        return _lrelu(hh * (g2 * _BN) + be2)

    ip = co(x[:, 0:1], cp_ref)
    il = co(x[:, 1:2], cl_ref)

    mw1 = mth_ref[0:14, 0:32]
    mb1 = mth_ref[14:15, 0:32]
    mg1 = mth_ref[15:16, 0:32]
    mbe1 = mth_ref[16:17, 0:32]
    mw2 = mth_ref[0:32, 32:96]
    mb2 = mth_ref[17:18, 32:96]
    mg2 = mth_ref[18:19, 32:96]
    mbe2 = mth_ref[19:20, 32:96]
    hm = jnp.dot(x[:, 2:16], mw1, preferred_element_type=jnp.float32) + mb1
    hm = _lrelu(hm * (mg1 * _BN) + mbe1)
    hm = jnp.dot(hm, mw2, preferred_element_type=jnp.float32) + mb2
    im = _lrelu(hm * (mg2 * _BN) + mbe2)

    fw1 = fw1_ref[...]
    y = (jnp.dot(preds, fw1[0:64], preferred_element_type=jnp.float32)
         + jnp.dot(ip, fw1[64:128], preferred_element_type=jnp.float32)
         + jnp.dot(il, fw1[128:192], preferred_element_type=jnp.float32)
         + jnp.dot(im, fw1[192:256], preferred_element_type=jnp.float32)
         + fb1_ref[...])
    h = _lrelu(y * (fg1_ref[...] * _BN) + fbe1_ref[...])
    h = jnp.dot(h, fw2_ref[...], preferred_element_type=jnp.float32) + fb2_ref[...]
    h = _lrelu(h * (fg2_ref[...] * _BN) + fbe2_ref[...])
    sg = jax.nn.sigmoid((h * fw3_ref[...]).sum(-1, keepdims=True) + fb3_ref[0, 0])
    o_ref[...] = jnp.broadcast_to(sg, (sg.shape[0], 8))


def _pack_co(p):
    buf = jnp.zeros((36, 96), jnp.float32)
    buf = buf.at[0, 0:32].set(p["W1"][0])
    buf = buf.at[1, 0:32].set(p["b1"])
    buf = buf.at[2, 0:32].set(p["g1"])
    buf = buf.at[3, 0:32].set(p["be1"])
    buf = buf.at[4:36, 32:96].set(p["W2"])
    buf = buf.at[1, 32:96].set(p["b2"])
    buf = buf.at[2, 32:96].set(p["g2"])
    buf = buf.at[3, 32:96].set(p["be2"])
    return buf


def _pack_meth(p):
    buf = jnp.zeros((32, 96), jnp.float32)
    buf = buf.at[0:14, 0:32].set(p["W1"])
    buf = buf.at[14, 0:32].set(p["b1"])
    buf = buf.at[15, 0:32].set(p["g1"])
    buf = buf.at[16, 0:32].set(p["be1"])
    buf = buf.at[0:32, 32:96].set(p["W2"])
    buf = buf.at[17, 32:96].set(p["b2"])
    buf = buf.at[18, 32:96].set(p["g2"])
    buf = buf.at[19, 32:96].set(p["be2"])
    return buf


def kernel(interaction, edge_index, graph_matrix, params):
    pg = params["gat"]
    full = lambda i: (0, 0)

    xlf, xrf = pl.pallas_call(
        _nodes_body,
        grid=(5,),
        in_specs=[
            pl.BlockSpec((2000, 192), lambda i: (i, 0)),
            pl.BlockSpec((70, 96), full),
            pl.BlockSpec((70, 96), full),
            pl.BlockSpec((70, 96), full),
            pl.BlockSpec((192, 256), full),
            pl.BlockSpec((1, 256), full),
            pl.BlockSpec((192, 256), full),
            pl.BlockSpec((1, 256), full),
        ],
        out_specs=[
            pl.BlockSpec((2000, 256), lambda i: (i, 0)),
            pl.BlockSpec((2000, 256), lambda i: (i, 0)),
        ],
        out_shape=[
            jax.ShapeDtypeStruct((N, 256), jnp.float32),
            jax.ShapeDtypeStruct((N, 256), jnp.float32),
        ],
    )(graph_matrix, _pack_sub(params["sub_g"]), _pack_sub(params["sub_p"]),
      _pack_sub(params["sub_l"]), pg["Wl"], pg["bl"].reshape(1, 256),
      pg["Wr"], pg["br"].reshape(1, 256))

    src = edge_index[:, 0]
    dst = edge_index[:, 1]
    srcp = jnp.concatenate([src, jnp.zeros((EPAD - E,), jnp.int32)])
    dstp = jnp.concatenate([dst, jnp.full((EPAD - E,), N, jnp.int32)])

    mesh = plsc.VectorSubcoreMesh(core_axis_name="c", subcore_axis_name="s",
                                  num_cores=NC, num_subcores=NS)

    p4, den32 = pl.kernel(
        _sc_logits_body,
        out_type=(
            jax.ShapeDtypeStruct((4 * EPAD,), jnp.float32),
            jax.ShapeDtypeStruct((NW, 1, 4 * NDEN), jnp.float32),
        ),
        mesh=mesh,
        compiler_params=_SCP,
        scratch_types=[
            pltpu.VMEM((256,), jnp.float32),
            pltpu.VMEM((16,), jnp.int32),
            pltpu.VMEM((16,), jnp.int32),
            pltpu.VMEM((16, 256), jnp.float32),
            pltpu.VMEM((16, 256), jnp.float32),
            pltpu.VMEM((4, 16), jnp.float32),
            pltpu.VMEM((16,), jnp.float32),
            pltpu.VMEM((16,), jnp.float32),
            pltpu.VMEM((16,), jnp.float32),
            pltpu.VMEM((16,), jnp.float32),
            pltpu.VMEM((4 * NDEN,), jnp.float32),
            pltpu.SemaphoreType.DMA,
        ],
    )(xlf, xrf, srcp, dstp, pg["att"].reshape(256),
      jnp.zeros((4 * NDEN,), jnp.float32))

    dinv = pl.pallas_call(
        _denom_body,
        grid=(1,),
        in_specs=[pl.BlockSpec((NW, 4, NDEN), lambda i: (0, 0, 0))],
        out_specs=pl.BlockSpec((8, NDEN), lambda i: (0, 0)),
        out_shape=jax.ShapeDtypeStruct((8, NDEN), jnp.float32),
    )(den32.reshape(NW, 4, NDEN))

    attn4 = pl.kernel(
        _sc_alpha_body,
        out_type=jax.ShapeDtypeStruct((4 * EPAD,), jnp.float32),
        mesh=mesh,
        compiler_params=_SCP,
        scratch_types=[
            pltpu.VMEM((NDEN,), jnp.float32),
            pltpu.VMEM((NDEN,), jnp.float32),
            pltpu.VMEM((NDEN,), jnp.float32),
            pltpu.VMEM((NDEN,), jnp.float32),
            pltpu.VMEM((16,), jnp.int32),
            pltpu.VMEM((16,), jnp.float32),
            pltpu.VMEM((16,), jnp.float32),
            pltpu.SemaphoreType.DMA,
        ],
    )(p4, dinv.reshape(8 * NDEN), dstp)

    acc32 = pl.kernel(
        _sc_aggregate_body,
        out_type=jax.ShapeDtypeStruct((NW, 1, RPT * 256), jnp.float32),
        mesh=mesh,
        compiler_params=_SCP,
        scratch_types=[
            pltpu.VMEM((STRIP,), jnp.int32),
            pltpu.VMEM((STRIP,), jnp.int32),
            pltpu.VMEM((STRIP + 16,), jnp.int32),
            pltpu.VMEM((STRIP + 16,), jnp.int32),
            pltpu.VMEM((STRIP + 16,), jnp.int32),
            pltpu.VMEM((16,), jnp.int32),
            pltpu.VMEM((16,), jnp.int32),
            pltpu.VMEM((16,), jnp.int32),
            pltpu.VMEM((16,), jnp.int32),
            pltpu.VMEM((16,), jnp.int32),
            pltpu.VMEM((16,), jnp.float32),
            pltpu.VMEM((16,), jnp.float32),
            pltpu.VMEM((16,), jnp.float32),
            pltpu.VMEM((16,), jnp.float32),
            pltpu.VMEM((16, 256), jnp.float32),
            pltpu.VMEM((RPT * 256,), jnp.float32),
            pltpu.SemaphoreType.DMA,
        ],
    )(xlf, attn4, srcp, dstp, jnp.zeros((RPT * 256,), jnp.float32))

    qn = pl.pallas_call(
        _nodepost_body,
        grid=(2,),
        in_specs=[
            pl.BlockSpec((NDEN // 2, 256), lambda i: (i, 0)),
            pl.BlockSpec((1, 256), full),
            pl.BlockSpec((256, 128), full),
            pl.BlockSpec((1, 128), full),
            pl.BlockSpec((1, 128), full),
            pl.BlockSpec((1, 128), full),
            pl.BlockSpec((128, 128), full),
        ],
        out_specs=pl.BlockSpec((NDEN // 2, 128), lambda i: (i, 0)),
        out_shape=jax.ShapeDtypeStruct((NDEN, 128), jnp.float32),
    )(acc32.reshape(NDEN, 256), pg["bias"].reshape(1, 256),
      params["phi1"]["W"], params["phi1"]["b"].reshape(1, 128),
      params["phi1"]["g"].reshape(1, 128),
      params["phi1"]["be"].reshape(1, 128), params["rho1"]["W1"])

    g_edges = pl.kernel(
        _sc_gather_body,
        out_type=jax.ShapeDtypeStruct((EPAD, 128), jnp.float32),
        mesh=mesh,
        compiler_params=_SCP,
        scratch_types=[
            pltpu.VMEM((32,), jnp.int32),
            pltpu.VMEM((32,), jnp.int32),
            pltpu.VMEM((32, 128), jnp.float32),
            pltpu.VMEM((32, 128), jnp.float32),
            pltpu.VMEM((32, 128), jnp.float32),
            pltpu.SemaphoreType.DMA,
        ],
    )(qn, srcp, dstp)

    pr = params["rho1"]
    pf = params["fc"]
    preds8 = pl.pallas_call(
        _edge_body,
        grid=(125,),
        in_specs=[
            pl.BlockSpec((1280, 128), lambda i: (i, 0)),
            pl.BlockSpec((1280, 16), lambda i: (i, 0)),
            pl.BlockSpec((1, 128), full),
            pl.BlockSpec((1, 128), full),
            pl.BlockSpec((1, 128), full),
            pl.BlockSpec((128, 64), full),
            pl.BlockSpec((1, 64), full),
            pl.BlockSpec((1, 64), full),
            pl.BlockSpec((1, 64), full),
            pl.BlockSpec((36, 96), full),
            pl.BlockSpec((36, 96), full),
            pl.BlockSpec((32, 96), full),
            pl.BlockSpec((256, 64), full),
            pl.BlockSpec((1, 64), full),
            pl.BlockSpec((1, 64), full),
            pl.BlockSpec((1, 64), full),
            pl.BlockSpec((64, 32), full),
            pl.BlockSpec((1, 32), full),
            pl.BlockSpec((1, 32), full),
            pl.BlockSpec((1, 32), full),
            pl.BlockSpec((1, 32), full),
            pl.BlockSpec((1, 1), full),
        ],
        out_specs=pl.BlockSpec((1280, 8), lambda i: (i, 0)),
        out_shape=jax.ShapeDtypeStruct((E, 8), jnp.float32),
    )(g_edges[:E], interaction,
      pr["b1"].reshape(1, 128), pr["g1"].reshape(1, 128),
      pr["be1"].reshape(1, 128), pr["W2"], pr["b2"].reshape(1, 64),
      pr["g2"].reshape(1, 64), pr["be2"].reshape(1, 64),
      _pack_co(params["co_p"]), _pack_co(params["co_loc"]),
      _pack_meth(params["meth"]),
      pf["W1"], pf["b1"].reshape(1, 64), pf["g1"].reshape(1, 64),
      pf["be1"].reshape(1, 64), pf["W2"], pf["b2"].reshape(1, 32),
      pf["g2"].reshape(1, 32), pf["be2"].reshape(1, 32),
      pf["W3"].reshape(1, 32), pf["b3"].reshape(1, 1))

    alpha = attn4.reshape(4, EPAD)[:, :E].T
    attn = jnp.concatenate([alpha, alpha], axis=0)
    return preds8[:, 0], attn


# speedup vs baseline: 1.2062x; 1.1488x over previous
import jax
import jax.numpy as jnp
import numpy as np
from jax import lax
from jax.experimental import pallas as pl
from jax.experimental.pallas import tpu as pltpu
from jax.experimental.pallas import tpu_sc as plsc

H = 64
N = 10000
E = 160000
NC = 2
NS = 16
NW = NC * NS
NDEN = 10240
NPADT = NDEN + 16
EPAD = 163840
RPT = NDEN // NW
STRIP = 2048
CHB = 256

_BN = float(1.0 / np.sqrt(1.0 + 1e-5))
_SCP = pltpu.CompilerParams(needs_layout_passes=False)


def _lrelu(x, s=0.01):
    return jnp.where(x >= 0, x, s * x)


def _nodes_body(gm_ref, sg_ref, sp_ref, sl_ref, wl_ref, bl_ref, wr_ref,
                br_ref, xl_ref, xr_ref):
    x = gm_ref[...]

    def sub(xin, p_ref):
        w1 = p_ref[0:64, 0:32]
        b1 = p_ref[64:65, 0:32]
        g1 = p_ref[65:66, 0:32]
        be1 = p_ref[66:67, 0:32]
        w2 = p_ref[0:32, 32:96]
        b2 = p_ref[67:68, 32:96]
        g2 = p_ref[68:69, 32:96]
        be2 = p_ref[69:70, 32:96]
        h = jnp.dot(xin, w1, preferred_element_type=jnp.float32) + b1
        h = _lrelu(h * (g1 * _BN) + be1)
        h = jnp.dot(h, w2, preferred_element_type=jnp.float32) + b2
        return _lrelu(h * (g2 * _BN) + be2)

    gm = jnp.concatenate([
        sub(x[:, 0:64], sg_ref),
        sub(x[:, 64:128], sp_ref),
        sub(x[:, 128:192], sl_ref),
    ], axis=1)
    xl_ref[...] = jnp.dot(gm, wl_ref[...],
                          preferred_element_type=jnp.float32) + bl_ref[...]
    xr_ref[...] = jnp.dot(gm, wr_ref[...],
                          preferred_element_type=jnp.float32) + br_ref[...]


def _pack_sub(p):
    buf = jnp.zeros((70, 96), jnp.float32)
    w1 = p["W1"]
    buf = buf.at[0:64, 0:32].set(w1)
    buf = buf.at[64, 0:32].set(p["b1"])
    buf = buf.at[65, 0:32].set(p["g1"])
    buf = buf.at[66, 0:32].set(p["be1"])
    buf = buf.at[0:32, 32:96].set(p["W2"])
    buf = buf.at[67, 32:96].set(p["b2"])
    buf = buf.at[68, 32:96].set(p["g2"])
    buf = buf.at[69, 32:96].set(p["be2"])
    return buf


def _sc_logits_body(xl_hbm, xr_hbm, src_hbm, dst_hbm, att_hbm, zden_hbm,
                    p4_hbm, den_hbm,
                    att_v, slo, dlo, xs, xrow, logit_b, pb0, pb1, pb2, pb3,
                    dtab, sem):
    pbufs = (pb0, pb1, pb2, pb3)
    c = lax.axis_index("c")
    s = lax.axis_index("s")
    wid = c * NS + s
    epw = EPAD // NW
    nchunks = epw // 16

    pltpu.sync_copy(att_hbm, att_v)
    pltpu.sync_copy(zden_hbm, dtab)
    iota = lax.iota(jnp.int32, 16)
    lane0 = (iota == 0).astype(jnp.float32)
    last = iota == 15

    def chunk(i, carry):
        base = wid * epw + i * 16
        pltpu.sync_copy(src_hbm.at[pl.ds(base, 16)], slo)
        pltpu.sync_copy(dst_hbm.at[pl.ds(base, 16)], dlo)
        cp1 = pltpu.async_copy(xl_hbm.at[slo], xs, sem)
        cp2 = pltpu.async_copy(xr_hbm.at[dlo], xrow, sem)
        cp1.wait()
        cp2.wait()
        for e in range(16):
            for h in range(4):
                acc = jnp.zeros((16,), jnp.float32)
                for k in range(4):
                    col = h * 64 + k * 16
                    sv = xs[e, pl.ds(col, 16)] + xrow[e, pl.ds(col, 16)]
                    lr = jnp.where(sv >= 0, sv, 0.2 * sv)
                    acc = acc + lr * att_v[pl.ds(col, 16)]
                cum = jnp.cumsum(acc)
                plsc.store_scatter(
                    logit_b,
                    [jnp.full((16,), h, jnp.int32),
                     jnp.full((16,), e, jnp.int32)],
                    cum, mask=last)
        dv = dlo[...]
        pvs = []
        for h in range(4):
            pv = jnp.exp(logit_b[h, :])
            pbufs[h][...] = pv
            pvs.append(pv)
            pltpu.sync_copy(pbufs[h], p4_hbm.at[pl.ds(h * EPAD + base, 16)])
        for e in range(16):
            d = dv[e]
            for h in range(4):
                off = h * NDEN + d
                cur = dtab[pl.ds(off, 16)]
                dtab[pl.ds(off, 16)] = cur + pvs[h][e] * lane0
        return carry

    lax.fori_loop(0, nchunks, chunk, None)
    pltpu.sync_copy(dtab, den_hbm.at[wid, 0])


def _denom_body(den_ref, out_ref):
    acc = den_ref[0]
    for i in range(1, NW):
        acc = acc + den_ref[i]
    inv = 1.0 / (2.0 * acc + 1e-16)
    out_ref[...] = jnp.concatenate([inv, inv], axis=0)


def _sc_alpha_body(p4_hbm, dinv_hbm, dst_hbm, attn_hbm,
                   den0, den1, den2, den3, dst_b, pv_b, ab, sem):
    dens = (den0, den1, den2, den3)
    c = lax.axis_index("c")
    s = lax.axis_index("s")
    wid = c * NS + s
    epw = EPAD // NW
    nchunks = epw // CHB

    for h in range(4):
        pltpu.sync_copy(dinv_hbm.at[pl.ds(h * NDEN, NDEN)], dens[h])

    def chunk(i, carry):
        base = wid * epw + i * CHB
        pltpu.sync_copy(dst_hbm.at[pl.ds(base, CHB)], dst_b)
        for h in range(4):
            pltpu.sync_copy(p4_hbm.at[pl.ds(h * EPAD + base, CHB)], pv_b)
            for j in range(CHB // 16):
                dv = dst_b[pl.ds(j * 16, 16)]
                dinv = plsc.load_gather(dens[h], [dv])
                ab[pl.ds(j * 16, 16)] = pv_b[pl.ds(j * 16, 16)] * dinv
            pltpu.sync_copy(ab, attn_hbm.at[pl.ds(h * EPAD + base, CHB)])
        return carry

    lax.fori_loop(0, nchunks, chunk, None)


def _sc_aggregate_body(xl_hbm, al_hbm, src_hbm, dst_hbm, zacc_hbm,
                       out_hbm,
                       dstrip, sstrip,
                       ls, ld, lidx, idx_b, ib0, ib1, ib2, ib3,
                       av0, av1, av2, av3, rows, acc, sem):
    c = lax.axis_index("c")
    s = lax.axis_index("s")
    wid = c * NS + s
    lo = wid * RPT
    hi = lo + RPT
    nstrips = EPAD // STRIP
    iota = lax.iota(jnp.int32, 16)

    pltpu.sync_copy(zacc_hbm, acc)

    def strip_fn(t, carry):
        sbase = t * STRIP
        pltpu.sync_copy(dst_hbm.at[pl.ds(sbase, STRIP)], dstrip)
        pltpu.sync_copy(src_hbm.at[pl.ds(sbase, STRIP)], sstrip)

        def grp(j, cnt):
            dv = dstrip[pl.ds(j * 16, 16)]
            m = jnp.logical_and(dv >= lo, dv < hi)
            npc = plsc.all_reduce_population_count(m)[0]

            @pl.when(npc > 0)
            def _():
                plsc.store_compressed(ld.at[pl.ds(cnt, 16)], dv, mask=m)
                plsc.store_compressed(ls.at[pl.ds(cnt, 16)],
                                      sstrip[pl.ds(j * 16, 16)], mask=m)
                plsc.store_compressed(lidx.at[pl.ds(cnt, 16)],
                                      sbase + j * 16 + iota, mask=m)

            return cnt + npc

        cnt = lax.fori_loop(0, STRIP // 16, grp, jnp.int32(0))
        ld[pl.ds(cnt, 16)] = jnp.full((16,), lo, jnp.int32)
        ls[pl.ds(cnt, 16)] = jnp.zeros((16,), jnp.int32)
        lidx[pl.ds(cnt, 16)] = jnp.zeros((16,), jnp.int32)

        def agg(k, carry2):
            kb = k * 16
            idx_b[...] = ls[pl.ds(kb, 16)]
            ev = lidx[pl.ds(kb, 16)]
            ib0[...] = ev
            ib1[...] = ev + EPAD
            ib2[...] = ev + 2 * EPAD
            ib3[...] = ev + 3 * EPAD
            cpr = pltpu.async_copy(xl_hbm.at[idx_b], rows, sem)
            c0 = pltpu.async_copy(al_hbm.at[ib0], av0, sem)
            c1 = pltpu.async_copy(al_hbm.at[ib1], av1, sem)
            c2 = pltpu.async_copy(al_hbm.at[ib2], av2, sem)
            c3 = pltpu.async_copy(al_hbm.at[ib3], av3, sem)
            cpr.wait()
            c0.wait()
            c1.wait()
            c2.wait()
            c3.wait()
            mk = ((kb + iota) < cnt).astype(jnp.float32)
            dm = ld[pl.ds(kb, 16)] - lo
            a0 = av0[...] * mk
            a1 = av1[...] * mk
            a2 = av2[...] * mk
            a3 = av3[...] * mk
            for e in range(16):
                d = dm[e]
                rbase = d * 256
                ws = (a0[e], a1[e], a2[e], a3[e])
                for k16 in range(16):
                    off = rbase + k16 * 16
                    cur = acc[pl.ds(off, 16)]
                    acc[pl.ds(off, 16)] = (
                        cur + rows[e, pl.ds(k16 * 16, 16)] * ws[k16 // 4])
            return carry2

        lax.fori_loop(0, (cnt + 15) // 16, agg, None)
        return carry

    lax.fori_loop(0, nstrips, strip_fn, None)
    pltpu.sync_copy(acc, out_hbm.at[wid, 0])


def _nodepost_body(acc_ref, bias_ref, pw_ref, pb_ref, pg_ref, pbe_ref,
                   rw1_ref, qn_ref):
    h1 = 2.0 * acc_ref[...] + bias_ref[...]
    p = jnp.dot(h1, pw_ref[...], preferred_element_type=jnp.float32) + pb_ref[...]
    p = _lrelu(p * (pg_ref[...] * _BN) + pbe_ref[...])
    qn_ref[...] = jnp.dot(p, rw1_ref[...], preferred_element_type=jnp.float32)


def _sc_gather_body(qn_hbm, src_hbm, dst_hbm, g_hbm,
                    src_b, dst_b, rs, rd, gsum, sem):
    c = lax.axis_index("c")
    s = lax.axis_index("s")
    wid = c * NS + s
    epw = EPAD // NW
    nchunks = epw // 32

    def chunk(i, carry):
        base = wid * epw + i * 32
        pltpu.sync_copy(src_hbm.at[pl.ds(base, 32)], src_b)
        pltpu.sync_copy(dst_hbm.at[pl.ds(base, 32)], dst_b)
        cp1 = pltpu.async_copy(qn_hbm.at[src_b], rs, sem)
        cp2 = pltpu.async_copy(qn_hbm.at[dst_b], rd, sem)
        cp1.wait()
        cp2.wait()
        for e in range(32):
            for k in range(8):
                gsum[e, pl.ds(k * 16, 16)] = (rs[e, pl.ds(k * 16, 16)]
                                              + rd[e, pl.ds(k * 16, 16)])
        pltpu.sync_copy(gsum, g_hbm.at[pl.ds(base, 32)])
        return carry

    lax.fori_loop(0, nchunks, chunk, None)


def _edge_body(g_ref, x_ref, rb1_ref, rg1_ref, rbe1_ref, rw2_ref, rb2_ref,
               rg2_ref, rbe2_ref, cp_ref, cl_ref, mth_ref, fw1_ref, fb1_ref,
               fg1_ref, fbe1_ref, fw2_ref, fb2_ref, fg2_ref, fbe2_ref,
               fw3_ref, fb3_ref, o_ref):
    g = g_ref[...]
    h = _lrelu((g + rb1_ref[...]) * (rg1_ref[...] * _BN) + rbe1_ref[...])
    h = jnp.dot(h, rw2_ref[...], preferred_element_type=jnp.float32) + rb2_ref[...]
    preds = _lrelu(h * (rg2_ref[...] * _BN) + rbe2_ref[...])

    x = x_ref[...]

    def co(xcol, p_ref):
        w1 = p_ref[0:1, 0:32]
        b1 = p_ref[1:2, 0:32]
        g1 = p_ref[2:3, 0:32]
        be1 = p_ref[3:4, 0:32]
        w2 = p_ref[4:36, 32:96]
        b2 = p_ref[1:2, 32:96]
        g2 = p_ref[2:3, 32:96]
        be2 = p_ref[3:4, 32:96]
        hh = xcol * w1 + b1
        hh = _lrelu(hh * (g1 * _BN) + be1)
        hh = jnp.dot(hh, w2, preferred_element_type=jnp.float32) + b2
        return _lrelu(hh * (g2 * _BN) + be2)

    ip = co(x[:, 0:1], cp_ref)
    il = co(x[:, 1:2], cl_ref)

    mw1 = mth_ref[0:14, 0:32]
    mb1 = mth_ref[14:15, 0:32]
    mg1 = mth_ref[15:16, 0:32]
    mbe1 = mth_ref[16:17, 0:32]
    mw2 = mth_ref[0:32, 32:96]
    mb2 = mth_ref[17:18, 32:96]
    mg2 = mth_ref[18:19, 32:96]
    mbe2 = mth_ref[19:20, 32:96]
    hm = jnp.dot(x[:, 2:16], mw1, preferred_element_type=jnp.float32) + mb1
    hm = _lrelu(hm * (mg1 * _BN) + mbe1)
    hm = jnp.dot(hm, mw2, preferred_element_type=jnp.float32) + mb2
    im = _lrelu(hm * (mg2 * _BN) + mbe2)

    fw1 = fw1_ref[...]
    y = (jnp.dot(preds, fw1[0:64], preferred_element_type=jnp.float32)
         + jnp.dot(ip, fw1[64:128], preferred_element_type=jnp.float32)
         + jnp.dot(il, fw1[128:192], preferred_element_type=jnp.float32)
         + jnp.dot(im, fw1[192:256], preferred_element_type=jnp.float32)
         + fb1_ref[...])
    h = _lrelu(y * (fg1_ref[...] * _BN) + fbe1_ref[...])
    h = jnp.dot(h, fw2_ref[...], preferred_element_type=jnp.float32) + fb2_ref[...]
    h = _lrelu(h * (fg2_ref[...] * _BN) + fbe2_ref[...])
    sg = jax.nn.sigmoid((h * fw3_ref[...]).sum(-1, keepdims=True) + fb3_ref[0, 0])
    o_ref[...] = jnp.broadcast_to(sg, (sg.shape[0], 8))


def _pack_co(p):
    buf = jnp.zeros((36, 96), jnp.float32)
    buf = buf.at[0, 0:32].set(p["W1"][0])
    buf = buf.at[1, 0:32].set(p["b1"])
    buf = buf.at[2, 0:32].set(p["g1"])
    buf = buf.at[3, 0:32].set(p["be1"])
    buf = buf.at[4:36, 32:96].set(p["W2"])
    buf = buf.at[1, 32:96].set(p["b2"])
    buf = buf.at[2, 32:96].set(p["g2"])
    buf = buf.at[3, 32:96].set(p["be2"])
    return buf


def _pack_meth(p):
    buf = jnp.zeros((32, 96), jnp.float32)
    buf = buf.at[0:14, 0:32].set(p["W1"])
    buf = buf.at[14, 0:32].set(p["b1"])
    buf = buf.at[15, 0:32].set(p["g1"])
    buf = buf.at[16, 0:32].set(p["be1"])
    buf = buf.at[0:32, 32:96].set(p["W2"])
    buf = buf.at[17, 32:96].set(p["b2"])
    buf = buf.at[18, 32:96].set(p["g2"])
    buf = buf.at[19, 32:96].set(p["be2"])
    return buf


def kernel(interaction, edge_index, graph_matrix, params):
    pg = params["gat"]
    full = lambda i: (0, 0)

    xlf, xrf = pl.pallas_call(
        _nodes_body,
        grid=(5,),
        in_specs=[
            pl.BlockSpec((2000, 192), lambda i: (i, 0)),
            pl.BlockSpec((70, 96), full),
            pl.BlockSpec((70, 96), full),
            pl.BlockSpec((70, 96), full),
            pl.BlockSpec((192, 256), full),
            pl.BlockSpec((1, 256), full),
            pl.BlockSpec((192, 256), full),
            pl.BlockSpec((1, 256), full),
        ],
        out_specs=[
            pl.BlockSpec((2000, 256), lambda i: (i, 0)),
            pl.BlockSpec((2000, 256), lambda i: (i, 0)),
        ],
        out_shape=[
            jax.ShapeDtypeStruct((N, 256), jnp.float32),
            jax.ShapeDtypeStruct((N, 256), jnp.float32),
        ],
    )(graph_matrix, _pack_sub(params["sub_g"]), _pack_sub(params["sub_p"]),
      _pack_sub(params["sub_l"]), pg["Wl"], pg["bl"].reshape(1, 256),
      pg["Wr"], pg["br"].reshape(1, 256))

    src = edge_index[:, 0]
    dst = edge_index[:, 1]
    srcp = jnp.concatenate([src, jnp.zeros((EPAD - E,), jnp.int32)])
    dstp = jnp.concatenate([dst, jnp.full((EPAD - E,), N, jnp.int32)])

    mesh = plsc.VectorSubcoreMesh(core_axis_name="c", subcore_axis_name="s",
                                  num_cores=NC, num_subcores=NS)

    p4, den32 = pl.kernel(
        _sc_logits_body,
        out_type=(
            jax.ShapeDtypeStruct((4 * EPAD,), jnp.float32),
            jax.ShapeDtypeStruct((NW, 1, 4 * NDEN), jnp.float32),
        ),
        mesh=mesh,
        compiler_params=_SCP,
        scratch_types=[
            pltpu.VMEM((256,), jnp.float32),
            pltpu.VMEM((16,), jnp.int32),
            pltpu.VMEM((16,), jnp.int32),
            pltpu.VMEM((16, 256), jnp.float32),
            pltpu.VMEM((16, 256), jnp.float32),
            pltpu.VMEM((4, 16), jnp.float32),
            pltpu.VMEM((16,), jnp.float32),
            pltpu.VMEM((16,), jnp.float32),
            pltpu.VMEM((16,), jnp.float32),
            pltpu.VMEM((16,), jnp.float32),
            pltpu.VMEM((4 * NDEN,), jnp.float32),
            pltpu.SemaphoreType.DMA,
        ],
    )(xlf, xrf, srcp, dstp, pg["att"].reshape(256),
      jnp.zeros((4 * NDEN,), jnp.float32))

    dinv = pl.pallas_call(
        _denom_body,
        grid=(1,),
        in_specs=[pl.BlockSpec((NW, 4, NDEN), lambda i: (0, 0, 0))],
        out_specs=pl.BlockSpec((8, NDEN), lambda i: (0, 0)),
        out_shape=jax.ShapeDtypeStruct((8, NDEN), jnp.float32),
    )(den32.reshape(NW, 4, NDEN))

    attn4 = pl.kernel(
        _sc_alpha_body,
        out_type=jax.ShapeDtypeStruct((4 * EPAD,), jnp.float32),
        mesh=mesh,
        compiler_params=_SCP,
        scratch_types=[
            pltpu.VMEM((NDEN,), jnp.float32),
            pltpu.VMEM((NDEN,), jnp.float32),
            pltpu.VMEM((NDEN,), jnp.float32),
            pltpu.VMEM((NDEN,), jnp.float32),
            pltpu.VMEM((CHB,), jnp.int32),
            pltpu.VMEM((CHB,), jnp.float32),
            pltpu.VMEM((CHB,), jnp.float32),
            pltpu.SemaphoreType.DMA,
        ],
    )(p4, dinv.reshape(8 * NDEN), dstp)

    acc32 = pl.kernel(
        _sc_aggregate_body,
        out_type=jax.ShapeDtypeStruct((NW, 1, RPT * 256), jnp.float32),
        mesh=mesh,
        compiler_params=_SCP,
        scratch_types=[
            pltpu.VMEM((STRIP,), jnp.int32),
            pltpu.VMEM((STRIP,), jnp.int32),
            pltpu.VMEM((STRIP + 16,), jnp.int32),
            pltpu.VMEM((STRIP + 16,), jnp.int32),
            pltpu.VMEM((STRIP + 16,), jnp.int32),
            pltpu.VMEM((16,), jnp.int32),
            pltpu.VMEM((16,), jnp.int32),
            pltpu.VMEM((16,), jnp.int32),
            pltpu.VMEM((16,), jnp.int32),
            pltpu.VMEM((16,), jnp.int32),
            pltpu.VMEM((16,), jnp.float32),
            pltpu.VMEM((16,), jnp.float32),
            pltpu.VMEM((16,), jnp.float32),
            pltpu.VMEM((16,), jnp.float32),
            pltpu.VMEM((16, 256), jnp.float32),
            pltpu.VMEM((RPT * 256,), jnp.float32),
            pltpu.SemaphoreType.DMA,
        ],
    )(xlf, attn4, srcp, dstp, jnp.zeros((RPT * 256,), jnp.float32))

    qn = pl.pallas_call(
        _nodepost_body,
        grid=(2,),
        in_specs=[
            pl.BlockSpec((NDEN // 2, 256), lambda i: (i, 0)),
            pl.BlockSpec((1, 256), full),
            pl.BlockSpec((256, 128), full),
            pl.BlockSpec((1, 128), full),
            pl.BlockSpec((1, 128), full),
            pl.BlockSpec((1, 128), full),
            pl.BlockSpec((128, 128), full),
        ],
        out_specs=pl.BlockSpec((NDEN // 2, 128), lambda i: (i, 0)),
        out_shape=jax.ShapeDtypeStruct((NDEN, 128), jnp.float32),
    )(acc32.reshape(NDEN, 256), pg["bias"].reshape(1, 256),
      params["phi1"]["W"], params["phi1"]["b"].reshape(1, 128),
      params["phi1"]["g"].reshape(1, 128),
      params["phi1"]["be"].reshape(1, 128), params["rho1"]["W1"])

    g_edges = pl.kernel(
        _sc_gather_body,
        out_type=jax.ShapeDtypeStruct((EPAD, 128), jnp.float32),
        mesh=mesh,
        compiler_params=_SCP,
        scratch_types=[
            pltpu.VMEM((32,), jnp.int32),
            pltpu.VMEM((32,), jnp.int32),
            pltpu.VMEM((32, 128), jnp.float32),
            pltpu.VMEM((32, 128), jnp.float32),
            pltpu.VMEM((32, 128), jnp.float32),
            pltpu.SemaphoreType.DMA,
        ],
    )(qn, srcp, dstp)

    pr = params["rho1"]
    pf = params["fc"]
    preds8 = pl.pallas_call(
        _edge_body,
        grid=(125,),
        in_specs=[
            pl.BlockSpec((1280, 128), lambda i: (i, 0)),
            pl.BlockSpec((1280, 16), lambda i: (i, 0)),
            pl.BlockSpec((1, 128), full),
            pl.BlockSpec((1, 128), full),
            pl.BlockSpec((1, 128), full),
            pl.BlockSpec((128, 64), full),
            pl.BlockSpec((1, 64), full),
            pl.BlockSpec((1, 64), full),
            pl.BlockSpec((1, 64), full),
            pl.BlockSpec((36, 96), full),
            pl.BlockSpec((36, 96), full),
            pl.BlockSpec((32, 96), full),
            pl.BlockSpec((256, 64), full),
            pl.BlockSpec((1, 64), full),
            pl.BlockSpec((1, 64), full),
            pl.BlockSpec((1, 64), full),
            pl.BlockSpec((64, 32), full),
            pl.BlockSpec((1, 32), full),
            pl.BlockSpec((1, 32), full),
            pl.BlockSpec((1, 32), full),
            pl.BlockSpec((1, 32), full),
            pl.BlockSpec((1, 1), full),
        ],
        out_specs=pl.BlockSpec((1280, 8), lambda i: (i, 0)),
        out_shape=jax.ShapeDtypeStruct((E, 8), jnp.float32),
    )(g_edges[:E], interaction,
      pr["b1"].reshape(1, 128), pr["g1"].reshape(1, 128),
      pr["be1"].reshape(1, 128), pr["W2"], pr["b2"].reshape(1, 64),
      pr["g2"].reshape(1, 64), pr["be2"].reshape(1, 64),
      _pack_co(params["co_p"]), _pack_co(params["co_loc"]),
      _pack_meth(params["meth"]),
      pf["W1"], pf["b1"].reshape(1, 64), pf["g1"].reshape(1, 64),
      pf["be1"].reshape(1, 64), pf["W2"], pf["b2"].reshape(1, 32),
      pf["g2"].reshape(1, 32), pf["be2"].reshape(1, 32),
      pf["W3"].reshape(1, 32), pf["b3"].reshape(1, 1))

    alpha = attn4.reshape(4, EPAD)[:, :E].T
    attn = jnp.concatenate([alpha, alpha], axis=0)
    return preds8[:, 0], attn


# speedup vs baseline: 1.3033x; 1.0805x over previous
import jax
import jax.numpy as jnp
import numpy as np
from jax import lax
from jax.experimental import pallas as pl
from jax.experimental.pallas import tpu as pltpu
from jax.experimental.pallas import tpu_sc as plsc

H = 64
N = 10000
E = 160000
NC = 2
NS = 16
NW = NC * NS
NDEN = 10240
NPADT = NDEN + 16
EPAD = 163840
RPT = NDEN // NW
STRIP = 2048
CHB = 256

_BN = float(1.0 / np.sqrt(1.0 + 1e-5))
_SCP = pltpu.CompilerParams(needs_layout_passes=False)


def _lrelu(x, s=0.01):
    return jnp.where(x >= 0, x, s * x)


def _nodes_body(gm_ref, sg_ref, sp_ref, sl_ref, wl_ref, bl_ref, wr_ref,
                br_ref, xl_ref, xr_ref):
    x = gm_ref[...]

    def sub(xin, p_ref):
        w1 = p_ref[0:64, 0:32]
        b1 = p_ref[64:65, 0:32]
        g1 = p_ref[65:66, 0:32]
        be1 = p_ref[66:67, 0:32]
        w2 = p_ref[0:32, 32:96]
        b2 = p_ref[67:68, 32:96]
        g2 = p_ref[68:69, 32:96]
        be2 = p_ref[69:70, 32:96]
        h = jnp.dot(xin, w1, preferred_element_type=jnp.float32) + b1
        h = _lrelu(h * (g1 * _BN) + be1)
        h = jnp.dot(h, w2, preferred_element_type=jnp.float32) + b2
        return _lrelu(h * (g2 * _BN) + be2)

    gm = jnp.concatenate([
        sub(x[:, 0:64], sg_ref),
        sub(x[:, 64:128], sp_ref),
        sub(x[:, 128:192], sl_ref),
    ], axis=1)
    xl_ref[...] = jnp.dot(gm, wl_ref[...],
                          preferred_element_type=jnp.float32) + bl_ref[...]
    xr_ref[...] = jnp.dot(gm, wr_ref[...],
                          preferred_element_type=jnp.float32) + br_ref[...]


def _pack_sub(p):
    buf = jnp.zeros((70, 96), jnp.float32)
    w1 = p["W1"]
    buf = buf.at[0:64, 0:32].set(w1)
    buf = buf.at[64, 0:32].set(p["b1"])
    buf = buf.at[65, 0:32].set(p["g1"])
    buf = buf.at[66, 0:32].set(p["be1"])
    buf = buf.at[0:32, 32:96].set(p["W2"])
    buf = buf.at[67, 32:96].set(p["b2"])
    buf = buf.at[68, 32:96].set(p["g2"])
    buf = buf.at[69, 32:96].set(p["be2"])
    return buf


def _sc_logits_body(xl_hbm, xr_hbm, src_hbm, dst_hbm, att_hbm, zden_hbm,
                    p4_hbm, den_hbm,
                    att_v, slo, dlo, sla, dla, xs, xrow, logit_b,
                    pb0, pb1, pb2, pb3, dtab, sem):
    pbufs = (pb0, pb1, pb2, pb3)
    c = lax.axis_index("c")
    s = lax.axis_index("s")
    wid = c * NS + s
    epw = EPAD // NW
    nchunks = epw // 16

    pltpu.sync_copy(att_hbm, att_v)
    pltpu.sync_copy(zden_hbm, dtab)
    pltpu.sync_copy(src_hbm.at[pl.ds(wid * epw, epw)], sla)
    pltpu.sync_copy(dst_hbm.at[pl.ds(wid * epw, epw)], dla)
    iota = lax.iota(jnp.int32, 16)
    lane0 = (iota == 0).astype(jnp.float32)
    last = iota == 15

    def chunk(i, carry):
        base = wid * epw + i * 16
        slo[...] = sla[pl.ds(i * 16, 16)]
        dlo[...] = dla[pl.ds(i * 16, 16)]
        cp1 = pltpu.async_copy(xl_hbm.at[slo], xs, sem)
        cp2 = pltpu.async_copy(xr_hbm.at[dlo], xrow, sem)
        cp1.wait()
        cp2.wait()
        for e in range(16):
            for h in range(4):
                acc = jnp.zeros((16,), jnp.float32)
                for k in range(4):
                    col = h * 64 + k * 16
                    sv = xs[e, pl.ds(col, 16)] + xrow[e, pl.ds(col, 16)]
                    lr = jnp.where(sv >= 0, sv, 0.2 * sv)
                    acc = acc + lr * att_v[pl.ds(col, 16)]
                cum = jnp.cumsum(acc)
                plsc.store_scatter(
                    logit_b,
                    [jnp.full((16,), h, jnp.int32),
                     jnp.full((16,), e, jnp.int32)],
                    cum, mask=last)
        dv = dlo[...]
        pvs = []
        for h in range(4):
            pv = jnp.exp(logit_b[h, :])
            pbufs[h][...] = pv
            pvs.append(pv)
            pltpu.sync_copy(pbufs[h], p4_hbm.at[pl.ds(h * EPAD + base, 16)])
        for e in range(16):
            d = dv[e]
            for h in range(4):
                off = h * NDEN + d
                cur = dtab[pl.ds(off, 16)]
                dtab[pl.ds(off, 16)] = cur + pvs[h][e] * lane0
        return carry

    lax.fori_loop(0, nchunks, chunk, None)
    pltpu.sync_copy(dtab, den_hbm.at[wid, 0])


def _denom_body(den_ref, out_ref):
    acc = den_ref[0]
    for i in range(1, NW):
        acc = acc + den_ref[i]
    inv = 1.0 / (2.0 * acc + 1e-16)
    out_ref[...] = jnp.concatenate([inv, inv], axis=0)


def _sc_alpha_body(p4_hbm, dinv_hbm, dst_hbm, attn_hbm,
                   den0, den1, den2, den3, dst_b, pv_b, ab, sem):
    dens = (den0, den1, den2, den3)
    c = lax.axis_index("c")
    s = lax.axis_index("s")
    wid = c * NS + s
    epw = EPAD // NW
    nchunks = epw // CHB

    for h in range(4):
        pltpu.sync_copy(dinv_hbm.at[pl.ds(h * NDEN, NDEN)], dens[h])

    def chunk(i, carry):
        base = wid * epw + i * CHB
        pltpu.sync_copy(dst_hbm.at[pl.ds(base, CHB)], dst_b)
        for h in range(4):
            pltpu.sync_copy(p4_hbm.at[pl.ds(h * EPAD + base, CHB)], pv_b)
            for j in range(CHB // 16):
                dv = dst_b[pl.ds(j * 16, 16)]
                dinv = plsc.load_gather(dens[h], [dv])
                ab[pl.ds(j * 16, 16)] = pv_b[pl.ds(j * 16, 16)] * dinv
            pltpu.sync_copy(ab, attn_hbm.at[pl.ds(h * EPAD + base, CHB)])
        return carry

    lax.fori_loop(0, nchunks, chunk, None)


def _sc_aggregate_body(xl_hbm, al_hbm, src_hbm, dst_hbm, zacc_hbm,
                       out_hbm,
                       dstrip, sstrip,
                       ls, ld, lidx, idx_b, ib0, ib1, ib2, ib3,
                       av0, av1, av2, av3, rows, acc, sem):
    c = lax.axis_index("c")
    s = lax.axis_index("s")
    wid = c * NS + s
    lo = wid * RPT
    hi = lo + RPT
    nstrips = EPAD // STRIP
    iota = lax.iota(jnp.int32, 16)

    pltpu.sync_copy(zacc_hbm, acc)

    def strip_fn(t, carry):
        sbase = t * STRIP
        pltpu.sync_copy(dst_hbm.at[pl.ds(sbase, STRIP)], dstrip)
        pltpu.sync_copy(src_hbm.at[pl.ds(sbase, STRIP)], sstrip)

        def grp(j, cnt):
            dv = dstrip[pl.ds(j * 16, 16)]
            m = jnp.logical_and(dv >= lo, dv < hi)
            npc = plsc.all_reduce_population_count(m)[0]

            @pl.when(npc > 0)
            def _():
                plsc.store_compressed(ld.at[pl.ds(cnt, 16)], dv, mask=m)
                plsc.store_compressed(ls.at[pl.ds(cnt, 16)],
                                      sstrip[pl.ds(j * 16, 16)], mask=m)
                plsc.store_compressed(lidx.at[pl.ds(cnt, 16)],
                                      sbase + j * 16 + iota, mask=m)

            return cnt + npc

        cnt = lax.fori_loop(0, STRIP // 16, grp, jnp.int32(0))
        ld[pl.ds(cnt, 16)] = jnp.full((16,), lo, jnp.int32)
        ls[pl.ds(cnt, 16)] = jnp.zeros((16,), jnp.int32)
        lidx[pl.ds(cnt, 16)] = jnp.zeros((16,), jnp.int32)

        def agg(k, carry2):
            kb = k * 16
            idx_b[...] = ls[pl.ds(kb, 16)]
            ev = lidx[pl.ds(kb, 16)]
            ib0[...] = ev
            ib1[...] = ev + EPAD
            ib2[...] = ev + 2 * EPAD
            ib3[...] = ev + 3 * EPAD
            cpr = pltpu.async_copy(xl_hbm.at[idx_b], rows, sem)
            c0 = pltpu.async_copy(al_hbm.at[ib0], av0, sem)
            c1 = pltpu.async_copy(al_hbm.at[ib1], av1, sem)
            c2 = pltpu.async_copy(al_hbm.at[ib2], av2, sem)
            c3 = pltpu.async_copy(al_hbm.at[ib3], av3, sem)
            cpr.wait()
            c0.wait()
            c1.wait()
            c2.wait()
            c3.wait()
            mk = ((kb + iota) < cnt).astype(jnp.float32)
            dm = ld[pl.ds(kb, 16)] - lo
            a0 = av0[...] * mk
            a1 = av1[...] * mk
            a2 = av2[...] * mk
            a3 = av3[...] * mk
            for e in range(16):
                d = dm[e]
                rbase = d * 256
                ws = (a0[e], a1[e], a2[e], a3[e])
                for k16 in range(16):
                    off = rbase + k16 * 16
                    cur = acc[pl.ds(off, 16)]
                    acc[pl.ds(off, 16)] = (
                        cur + rows[e, pl.ds(k16 * 16, 16)] * ws[k16 // 4])
            return carry2

        lax.fori_loop(0, (cnt + 15) // 16, agg, None)
        return carry

    lax.fori_loop(0, nstrips, strip_fn, None)
    pltpu.sync_copy(acc, out_hbm.at[wid, 0])


def _nodepost_body(acc_ref, bias_ref, pw_ref, pb_ref, pg_ref, pbe_ref,
                   rw1_ref, qn_ref):
    h1 = 2.0 * acc_ref[...] + bias_ref[...]
    p = jnp.dot(h1, pw_ref[...], preferred_element_type=jnp.float32) + pb_ref[...]
    p = _lrelu(p * (pg_ref[...] * _BN) + pbe_ref[...])
    qn_ref[...] = jnp.dot(p, rw1_ref[...], preferred_element_type=jnp.float32)


def _sc_gather_body(qn_hbm, src_hbm, dst_hbm, g_hbm,
                    src_b, dst_b, sida, dida, rs, rd, gsum, sem):
    c = lax.axis_index("c")
    s = lax.axis_index("s")
    wid = c * NS + s
    epw = EPAD // NW
    nchunks = epw // 32

    pltpu.sync_copy(src_hbm.at[pl.ds(wid * epw, epw)], sida)
    pltpu.sync_copy(dst_hbm.at[pl.ds(wid * epw, epw)], dida)

    def chunk(i, carry):
        base = wid * epw + i * 32
        for j in range(2):
            src_b[pl.ds(j * 16, 16)] = sida[pl.ds(i * 32 + j * 16, 16)]
            dst_b[pl.ds(j * 16, 16)] = dida[pl.ds(i * 32 + j * 16, 16)]
        cp1 = pltpu.async_copy(qn_hbm.at[src_b], rs, sem)
        cp2 = pltpu.async_copy(qn_hbm.at[dst_b], rd, sem)
        cp1.wait()
        cp2.wait()
        for e in range(32):
            for k in range(8):
                gsum[e, pl.ds(k * 16, 16)] = (rs[e, pl.ds(k * 16, 16)]
                                              + rd[e, pl.ds(k * 16, 16)])
        pltpu.sync_copy(gsum, g_hbm.at[pl.ds(base, 32)])
        return carry

    lax.fori_loop(0, nchunks, chunk, None)


def _edge_body(g_ref, x_ref, rb1_ref, rg1_ref, rbe1_ref, rw2_ref, rb2_ref,
               rg2_ref, rbe2_ref, cp_ref, cl_ref, mth_ref, fw1_ref, fb1_ref,
               fg1_ref, fbe1_ref, fw2_ref, fb2_ref, fg2_ref, fbe2_ref,
               fw3_ref, fb3_ref, o_ref):
    g = g_ref[...]
    h = _lrelu((g + rb1_ref[...]) * (rg1_ref[...] * _BN) + rbe1_ref[...])
    h = jnp.dot(h, rw2_ref[...], preferred_element_type=jnp.float32) + rb2_ref[...]
    preds = _lrelu(h * (rg2_ref[...] * _BN) + rbe2_ref[...])

    x = x_ref[...]

    def co(xcol, p_ref):
        w1 = p_ref[0:1, 0:32]
        b1 = p_ref[1:2, 0:32]
        g1 = p_ref[2:3, 0:32]
        be1 = p_ref[3:4, 0:32]
        w2 = p_ref[4:36, 32:96]
        b2 = p_ref[1:2, 32:96]
        g2 = p_ref[2:3, 32:96]
        be2 = p_ref[3:4, 32:96]
        hh = xcol * w1 + b1
        hh = _lrelu(hh * (g1 * _BN) + be1)
        hh = jnp.dot(hh, w2, preferred_element_type=jnp.float32) + b2
        return _lrelu(hh * (g2 * _BN) + be2)

    ip = co(x[:, 0:1], cp_ref)
    il = co(x[:, 1:2], cl_ref)

    mw1 = mth_ref[0:14, 0:32]
    mb1 = mth_ref[14:15, 0:32]
    mg1 = mth_ref[15:16, 0:32]
    mbe1 = mth_ref[16:17, 0:32]
    mw2 = mth_ref[0:32, 32:96]
    mb2 = mth_ref[17:18, 32:96]
    mg2 = mth_ref[18:19, 32:96]
    mbe2 = mth_ref[19:20, 32:96]
    hm = jnp.dot(x[:, 2:16], mw1, preferred_element_type=jnp.float32) + mb1
    hm = _lrelu(hm * (mg1 * _BN) + mbe1)
    hm = jnp.dot(hm, mw2, preferred_element_type=jnp.float32) + mb2
    im = _lrelu(hm * (mg2 * _BN) + mbe2)

    fw1 = fw1_ref[...]
    y = (jnp.dot(preds, fw1[0:64], preferred_element_type=jnp.float32)
         + jnp.dot(ip, fw1[64:128], preferred_element_type=jnp.float32)
         + jnp.dot(il, fw1[128:192], preferred_element_type=jnp.float32)
         + jnp.dot(im, fw1[192:256], preferred_element_type=jnp.float32)
         + fb1_ref[...])
    h = _lrelu(y * (fg1_ref[...] * _BN) + fbe1_ref[...])
    h = jnp.dot(h, fw2_ref[...], preferred_element_type=jnp.float32) + fb2_ref[...]
    h = _lrelu(h * (fg2_ref[...] * _BN) + fbe2_ref[...])
    sg = jax.nn.sigmoid((h * fw3_ref[...]).sum(-1, keepdims=True) + fb3_ref[0, 0])
    o_ref[...] = jnp.broadcast_to(sg, (sg.shape[0], 8))


def _pack_co(p):
    buf = jnp.zeros((36, 96), jnp.float32)
    buf = buf.at[0, 0:32].set(p["W1"][0])
    buf = buf.at[1, 0:32].set(p["b1"])
    buf = buf.at[2, 0:32].set(p["g1"])
    buf = buf.at[3, 0:32].set(p["be1"])
    buf = buf.at[4:36, 32:96].set(p["W2"])
    buf = buf.at[1, 32:96].set(p["b2"])
    buf = buf.at[2, 32:96].set(p["g2"])
    buf = buf.at[3, 32:96].set(p["be2"])
    return buf


def _pack_meth(p):
    buf = jnp.zeros((32, 96), jnp.float32)
    buf = buf.at[0:14, 0:32].set(p["W1"])
    buf = buf.at[14, 0:32].set(p["b1"])
    buf = buf.at[15, 0:32].set(p["g1"])
    buf = buf.at[16, 0:32].set(p["be1"])
    buf = buf.at[0:32, 32:96].set(p["W2"])
    buf = buf.at[17, 32:96].set(p["b2"])
    buf = buf.at[18, 32:96].set(p["g2"])
    buf = buf.at[19, 32:96].set(p["be2"])
    return buf


def kernel(interaction, edge_index, graph_matrix, params):
    pg = params["gat"]
    full = lambda i: (0, 0)

    xlf, xrf = pl.pallas_call(
        _nodes_body,
        grid=(5,),
        in_specs=[
            pl.BlockSpec((2000, 192), lambda i: (i, 0)),
            pl.BlockSpec((70, 96), full),
            pl.BlockSpec((70, 96), full),
            pl.BlockSpec((70, 96), full),
            pl.BlockSpec((192, 256), full),
            pl.BlockSpec((1, 256), full),
            pl.BlockSpec((192, 256), full),
            pl.BlockSpec((1, 256), full),
        ],
        out_specs=[
            pl.BlockSpec((2000, 256), lambda i: (i, 0)),
            pl.BlockSpec((2000, 256), lambda i: (i, 0)),
        ],
        out_shape=[
            jax.ShapeDtypeStruct((N, 256), jnp.float32),
            jax.ShapeDtypeStruct((N, 256), jnp.float32),
        ],
    )(graph_matrix, _pack_sub(params["sub_g"]), _pack_sub(params["sub_p"]),
      _pack_sub(params["sub_l"]), pg["Wl"], pg["bl"].reshape(1, 256),
      pg["Wr"], pg["br"].reshape(1, 256))

    src = edge_index[:, 0]
    dst = edge_index[:, 1]
    srcp = jnp.concatenate([src, jnp.zeros((EPAD - E,), jnp.int32)])
    dstp = jnp.concatenate([dst, jnp.full((EPAD - E,), N, jnp.int32)])

    mesh = plsc.VectorSubcoreMesh(core_axis_name="c", subcore_axis_name="s",
                                  num_cores=NC, num_subcores=NS)

    p4, den32 = pl.kernel(
        _sc_logits_body,
        out_type=(
            jax.ShapeDtypeStruct((4 * EPAD,), jnp.float32),
            jax.ShapeDtypeStruct((NW, 1, 4 * NDEN), jnp.float32),
        ),
        mesh=mesh,
        compiler_params=_SCP,
        scratch_types=[
            pltpu.VMEM((256,), jnp.float32),
            pltpu.VMEM((16,), jnp.int32),
            pltpu.VMEM((16,), jnp.int32),
            pltpu.VMEM((EPAD // NW,), jnp.int32),
            pltpu.VMEM((EPAD // NW,), jnp.int32),
            pltpu.VMEM((16, 256), jnp.float32),
            pltpu.VMEM((16, 256), jnp.float32),
            pltpu.VMEM((4, 16), jnp.float32),
            pltpu.VMEM((16,), jnp.float32),
            pltpu.VMEM((16,), jnp.float32),
            pltpu.VMEM((16,), jnp.float32),
            pltpu.VMEM((16,), jnp.float32),
            pltpu.VMEM((4 * NDEN,), jnp.float32),
            pltpu.SemaphoreType.DMA,
        ],
    )(xlf, xrf, srcp, dstp, pg["att"].reshape(256),
      jnp.zeros((4 * NDEN,), jnp.float32))

    dinv = pl.pallas_call(
        _denom_body,
        grid=(1,),
        in_specs=[pl.BlockSpec((NW, 4, NDEN), lambda i: (0, 0, 0))],
        out_specs=pl.BlockSpec((8, NDEN), lambda i: (0, 0)),
        out_shape=jax.ShapeDtypeStruct((8, NDEN), jnp.float32),
    )(den32.reshape(NW, 4, NDEN))

    attn4 = pl.kernel(
        _sc_alpha_body,
        out_type=jax.ShapeDtypeStruct((4 * EPAD,), jnp.float32),
        mesh=mesh,
        compiler_params=_SCP,
        scratch_types=[
            pltpu.VMEM((NDEN,), jnp.float32),
            pltpu.VMEM((NDEN,), jnp.float32),
            pltpu.VMEM((NDEN,), jnp.float32),
            pltpu.VMEM((NDEN,), jnp.float32),
            pltpu.VMEM((CHB,), jnp.int32),
            pltpu.VMEM((CHB,), jnp.float32),
            pltpu.VMEM((CHB,), jnp.float32),
            pltpu.SemaphoreType.DMA,
        ],
    )(p4, dinv.reshape(8 * NDEN), dstp)

    acc32 = pl.kernel(
        _sc_aggregate_body,
        out_type=jax.ShapeDtypeStruct((NW, 1, RPT * 256), jnp.float32),
        mesh=mesh,
        compiler_params=_SCP,
        scratch_types=[
            pltpu.VMEM((STRIP,), jnp.int32),
            pltpu.VMEM((STRIP,), jnp.int32),
            pltpu.VMEM((STRIP + 16,), jnp.int32),
            pltpu.VMEM((STRIP + 16,), jnp.int32),
            pltpu.VMEM((STRIP + 16,), jnp.int32),
            pltpu.VMEM((16,), jnp.int32),
            pltpu.VMEM((16,), jnp.int32),
            pltpu.VMEM((16,), jnp.int32),
            pltpu.VMEM((16,), jnp.int32),
            pltpu.VMEM((16,), jnp.int32),
            pltpu.VMEM((16,), jnp.float32),
            pltpu.VMEM((16,), jnp.float32),
            pltpu.VMEM((16,), jnp.float32),
            pltpu.VMEM((16,), jnp.float32),
            pltpu.VMEM((16, 256), jnp.float32),
            pltpu.VMEM((RPT * 256,), jnp.float32),
            pltpu.SemaphoreType.DMA,
        ],
    )(xlf, attn4, srcp, dstp, jnp.zeros((RPT * 256,), jnp.float32))

    qn = pl.pallas_call(
        _nodepost_body,
        grid=(2,),
        in_specs=[
            pl.BlockSpec((NDEN // 2, 256), lambda i: (i, 0)),
            pl.BlockSpec((1, 256), full),
            pl.BlockSpec((256, 128), full),
            pl.BlockSpec((1, 128), full),
            pl.BlockSpec((1, 128), full),
            pl.BlockSpec((1, 128), full),
            pl.BlockSpec((128, 128), full),
        ],
        out_specs=pl.BlockSpec((NDEN // 2, 128), lambda i: (i, 0)),
        out_shape=jax.ShapeDtypeStruct((NDEN, 128), jnp.float32),
    )(acc32.reshape(NDEN, 256), pg["bias"].reshape(1, 256),
      params["phi1"]["W"], params["phi1"]["b"].reshape(1, 128),
      params["phi1"]["g"].reshape(1, 128),
      params["phi1"]["be"].reshape(1, 128), params["rho1"]["W1"])

    g_edges = pl.kernel(
        _sc_gather_body,
        out_type=jax.ShapeDtypeStruct((EPAD, 128), jnp.float32),
        mesh=mesh,
        compiler_params=_SCP,
        scratch_types=[
            pltpu.VMEM((32,), jnp.int32),
            pltpu.VMEM((32,), jnp.int32),
            pltpu.VMEM((EPAD // NW,), jnp.int32),
            pltpu.VMEM((EPAD // NW,), jnp.int32),
            pltpu.VMEM((32, 128), jnp.float32),
            pltpu.VMEM((32, 128), jnp.float32),
            pltpu.VMEM((32, 128), jnp.float32),
            pltpu.SemaphoreType.DMA,
        ],
    )(qn, srcp, dstp)

    pr = params["rho1"]
    pf = params["fc"]
    preds8 = pl.pallas_call(
        _edge_body,
        grid=(125,),
        in_specs=[
            pl.BlockSpec((1280, 128), lambda i: (i, 0)),
            pl.BlockSpec((1280, 16), lambda i: (i, 0)),
            pl.BlockSpec((1, 128), full),
            pl.BlockSpec((1, 128), full),
            pl.BlockSpec((1, 128), full),
            pl.BlockSpec((128, 64), full),
            pl.BlockSpec((1, 64), full),
            pl.BlockSpec((1, 64), full),
            pl.BlockSpec((1, 64), full),
            pl.BlockSpec((36, 96), full),
            pl.BlockSpec((36, 96), full),
            pl.BlockSpec((32, 96), full),
            pl.BlockSpec((256, 64), full),
            pl.BlockSpec((1, 64), full),
            pl.BlockSpec((1, 64), full),
            pl.BlockSpec((1, 64), full),
            pl.BlockSpec((64, 32), full),
            pl.BlockSpec((1, 32), full),
            pl.BlockSpec((1, 32), full),
            pl.BlockSpec((1, 32), full),
            pl.BlockSpec((1, 32), full),
            pl.BlockSpec((1, 1), full),
        ],
        out_specs=pl.BlockSpec((1280, 8), lambda i: (i, 0)),
        out_shape=jax.ShapeDtypeStruct((E, 8), jnp.float32),
    )(g_edges[:E], interaction,
      pr["b1"].reshape(1, 128), pr["g1"].reshape(1, 128),
      pr["be1"].reshape(1, 128), pr["W2"], pr["b2"].reshape(1, 64),
      pr["g2"].reshape(1, 64), pr["be2"].reshape(1, 64),
      _pack_co(params["co_p"]), _pack_co(params["co_loc"]),
      _pack_meth(params["meth"]),
      pf["W1"], pf["b1"].reshape(1, 64), pf["g1"].reshape(1, 64),
      pf["be1"].reshape(1, 64), pf["W2"], pf["b2"].reshape(1, 32),
      pf["g2"].reshape(1, 32), pf["be2"].reshape(1, 32),
      pf["W3"].reshape(1, 32), pf["b3"].reshape(1, 1))

    alpha = attn4.reshape(4, EPAD)[:, :E].T
    attn = jnp.concatenate([alpha, alpha], axis=0)
    return preds8[:, 0], attn
